# Initial kernel scaffold; baseline (speedup 1.0000x reference)
import jax
import jax.numpy as jnp
from jax.experimental import pallas as pl


def kernel(x, edge_index, W_proj, b_proj, g1, b1, Wq1, Wk1, Wv1, tau1, Wq2, Wk2, Wv2, tau2, Wq3, Wk3, Wv3, tau3, W_ctx, b_ctx, g2, b2):
    raise NotImplementedError("write your pallas kernel here")


# speedup vs baseline: 8.5377x; 8.5377x over previous
import functools

import jax
import jax.numpy as jnp
from jax import lax
from jax.experimental import pallas as pl
from jax.experimental.pallas import tpu as pltpu
from jax.experimental.pallas import tpu_sc as plsc

N = 10000
D = 128
H = 128
OUT = 768
E = 320000

NP = 10240
NTILES = 32
C = 128
EPT_CHUNKS = 81
EPT = EPT_CHUNKS * C
EP = NTILES * EPT
LN_EPS = 1e-5

_SC_PARAMS = pltpu.CompilerParams(needs_layout_passes=False)


def _mesh():
    return plsc.VectorSubcoreMesh(
        core_axis_name="c", subcore_axis_name="s", num_cores=2, num_subcores=16
    )


def _gelu(x):
    return 0.5 * x * (1.0 + lax.erf(x * (2.0 ** -0.5)))


def _proj_ln_body(x_ref, w_ref, b_ref, g_ref, bb_ref, o_ref):
    y = jnp.dot(x_ref[...], w_ref[...], preferred_element_type=jnp.float32)
    y = y + b_ref[...]
    mu = jnp.mean(y, axis=-1, keepdims=True)
    var = jnp.mean(jnp.square(y - mu), axis=-1, keepdims=True)
    o_ref[...] = (y - mu) * lax.rsqrt(var + LN_EPS) * g_ref[...] + bb_ref[...]


def _proj_ln(x, w, b, g, bb):
    BN = 1024
    return pl.pallas_call(
        _proj_ln_body,
        grid=(NP // BN,),
        in_specs=[
            pl.BlockSpec((BN, D), lambda i: (i, 0)),
            pl.BlockSpec((D, H), lambda i: (0, 0)),
            pl.BlockSpec((1, H), lambda i: (0, 0)),
            pl.BlockSpec((1, H), lambda i: (0, 0)),
            pl.BlockSpec((1, H), lambda i: (0, 0)),
        ],
        out_specs=pl.BlockSpec((BN, H), lambda i: (i, 0)),
        out_shape=jax.ShapeDtypeStruct((NP, H), jnp.float32),
    )(x, w, b, g, bb)


def _qkv1_body(tau_ref, h_ref, wq_ref, wk_ref, wv_ref, q_ref, k_ref, v_ref):
    h = h_ref[...]
    scale = 1.0 / jnp.maximum(tau_ref[0, 0], 0.001)
    q_ref[...] = jnp.dot(h, wq_ref[...], preferred_element_type=jnp.float32) * scale
    k_ref[...] = jnp.dot(h, wk_ref[...], preferred_element_type=jnp.float32)
    v_ref[...] = jnp.dot(h, wv_ref[...], preferred_element_type=jnp.float32)


def _qkv1(tau, h, wq, wk, wv):
    BN = 1024
    return pl.pallas_call(
        _qkv1_body,
        grid=(NP // BN,),
        in_specs=[
            pl.BlockSpec(memory_space=pltpu.SMEM),
            pl.BlockSpec((BN, H), lambda i: (i, 0)),
            pl.BlockSpec((H, H), lambda i: (0, 0)),
            pl.BlockSpec((H, H), lambda i: (0, 0)),
            pl.BlockSpec((H, H), lambda i: (0, 0)),
        ],
        out_specs=[
            pl.BlockSpec((BN, H), lambda i: (i, 0)),
            pl.BlockSpec((BN, H), lambda i: (i, 0)),
            pl.BlockSpec((BN, H), lambda i: (i, 0)),
        ],
        out_shape=[
            jax.ShapeDtypeStruct((NP, H), jnp.float32),
            jax.ShapeDtypeStruct((NP, H), jnp.float32),
            jax.ShapeDtypeStruct((NP, H), jnp.float32),
        ],
    )(tau, h, wq, wk, wv)


def _qkv2_body(tau_ref, a0_ref, a1_ref, wq_ref, wk_ref, wv_ref,
               h_ref, q_ref, k_ref, v_ref):
    h = _gelu(a0_ref[...] + a1_ref[...])
    h_ref[...] = h
    scale = 1.0 / jnp.maximum(tau_ref[0, 0], 0.001)
    q_ref[...] = jnp.dot(h, wq_ref[...], preferred_element_type=jnp.float32) * scale
    k_ref[...] = jnp.dot(h, wk_ref[...], preferred_element_type=jnp.float32)
    v_ref[...] = jnp.dot(h, wv_ref[...], preferred_element_type=jnp.float32)


def _qkv2(tau, a0, a1, wq, wk, wv):
    BN = 1024
    return pl.pallas_call(
        _qkv2_body,
        grid=(NP // BN,),
        in_specs=[
            pl.BlockSpec(memory_space=pltpu.SMEM),
            pl.BlockSpec((BN, H), lambda i: (i, 0)),
            pl.BlockSpec((BN, H), lambda i: (i, 0)),
            pl.BlockSpec((H, H), lambda i: (0, 0)),
            pl.BlockSpec((H, H), lambda i: (0, 0)),
            pl.BlockSpec((H, H), lambda i: (0, 0)),
        ],
        out_specs=[
            pl.BlockSpec((BN, H), lambda i: (i, 0)),
            pl.BlockSpec((BN, H), lambda i: (i, 0)),
            pl.BlockSpec((BN, H), lambda i: (i, 0)),
            pl.BlockSpec((BN, H), lambda i: (i, 0)),
        ],
        out_shape=[
            jax.ShapeDtypeStruct((NP, H), jnp.float32),
            jax.ShapeDtypeStruct((NP, H), jnp.float32),
            jax.ShapeDtypeStruct((NP, H), jnp.float32),
            jax.ShapeDtypeStruct((NP, H), jnp.float32),
        ],
    )(tau, a0, a1, wq, wk, wv)


def _colmax_body(p_ref, o_ref):
    o_ref[...] = jnp.max(p_ref[...], axis=0, keepdims=True)


def _colsum_body(p_ref, o_ref):
    o_ref[...] = jnp.sum(p_ref[...], axis=0, keepdims=True)


def _colreduce(p, body):
    BC = 1280
    return pl.pallas_call(
        body,
        grid=(NP // BC,),
        in_specs=[pl.BlockSpec((NTILES, BC), lambda i: (0, i))],
        out_specs=pl.BlockSpec((1, BC), lambda i: (0, i)),
        out_shape=jax.ShapeDtypeStruct((1, NP), jnp.float32),
    )(p)


def _final_body(h0_ref, h1_ref, h2_ref, a0_ref, a1_ref, w_ref, b_ref,
                g_ref, bb_ref, o_ref):
    h3 = _gelu(a0_ref[...] + a1_ref[...])
    w = w_ref[...]
    acc = jnp.dot(h0_ref[...], w[0:H], preferred_element_type=jnp.float32)
    acc = acc + jnp.dot(h1_ref[...], w[H:2 * H], preferred_element_type=jnp.float32)
    acc = acc + jnp.dot(h2_ref[...], w[2 * H:3 * H], preferred_element_type=jnp.float32)
    acc = acc + jnp.dot(h3, w[3 * H:4 * H], preferred_element_type=jnp.float32)
    y = _gelu(acc + b_ref[...])
    mu = jnp.mean(y, axis=-1, keepdims=True)
    var = jnp.mean(jnp.square(y - mu), axis=-1, keepdims=True)
    o_ref[...] = (y - mu) * lax.rsqrt(var + LN_EPS) * g_ref[...] + bb_ref[...]


def _final(h0, h1, h2, a0, a1, w, b, g, bb):
    BN = 512
    return pl.pallas_call(
        _final_body,
        grid=(NP // BN,),
        in_specs=[
            pl.BlockSpec((BN, H), lambda i: (i, 0)),
            pl.BlockSpec((BN, H), lambda i: (i, 0)),
            pl.BlockSpec((BN, H), lambda i: (i, 0)),
            pl.BlockSpec((BN, H), lambda i: (i, 0)),
            pl.BlockSpec((BN, H), lambda i: (i, 0)),
            pl.BlockSpec((4 * H, OUT), lambda i: (0, 0)),
            pl.BlockSpec((1, OUT), lambda i: (0, 0)),
            pl.BlockSpec((1, OUT), lambda i: (0, 0)),
            pl.BlockSpec((1, OUT), lambda i: (0, 0)),
        ],
        out_specs=pl.BlockSpec((BN, OUT), lambda i: (i, 0)),
        out_shape=jax.ShapeDtypeStruct((NP, OUT), jnp.float32),
    )(h0, h1, h2, a0, a1, w, b, g, bb)


@functools.partial(
    pl.kernel,
    out_type=[
        jax.ShapeDtypeStruct((EP,), jnp.float32),
        jax.ShapeDtypeStruct((NTILES, NP), jnp.float32),
    ],
    mesh=_mesh(),
    compiler_params=_SC_PARAMS,
    scratch_types=[
        pltpu.VMEM((C,), jnp.int32),
        pltpu.VMEM((C,), jnp.int32),
        pltpu.VMEM((C, D), jnp.float32),
        pltpu.VMEM((C, D), jnp.float32),
        pltpu.VMEM((C,), jnp.float32),
        pltpu.VMEM((NP,), jnp.float32),
        pltpu.SemaphoreType.DMA,
        pltpu.SemaphoreType.DMA,
    ],
)
def _s1(q_hbm, k_hbm, src_hbm, dst_hbm, score_hbm, pm_hbm,
        srcv, dstv, qrows, krows, scb, pm_l, sem1, sem2):
    cc = lax.axis_index("c")
    ss = lax.axis_index("s")
    wid = ss * 2 + cc
    base0 = wid * EPT
    lanes = lax.iota(jnp.int32, 16)

    neg = jnp.full((16,), -3.0e38, jnp.float32)

    def initb(i, _):
        pm_l[pl.ds(i * 16, 16)] = neg
        return 0

    lax.fori_loop(0, NP // 16, initb, 0)

    def chunk(ci, _):
        base = base0 + ci * C
        pltpu.sync_copy(src_hbm.at[pl.ds(base, C)], srcv)
        pltpu.sync_copy(dst_hbm.at[pl.ds(base, C)], dstv)
        cp1 = pltpu.async_copy(q_hbm.at[dstv], qrows, sem1)
        cp2 = pltpu.async_copy(k_hbm.at[srcv], krows, sem2)
        cp1.wait()
        cp2.wait()

        def grp(g, _):
            scv = jnp.zeros((16,), jnp.float32)
            for j in range(16):
                i = g * 16 + j
                a = qrows[i, pl.ds(0, 16)] * krows[i, pl.ds(0, 16)]
                for dd in range(1, D // 16):
                    a = a + (qrows[i, pl.ds(dd * 16, 16)]
                             * krows[i, pl.ds(dd * 16, 16)])
                scv = jnp.where(lanes == j, jnp.sum(a), scv)
            scb[pl.ds(g * 16, 16)] = scv
            dv = dstv[pl.ds(g * 16, 16)]

            def bodyw(_carry):
                cur = plsc.load_gather(pm_l, [dv])
                need = scv > cur
                plsc.store_scatter(pm_l, [dv], scv, mask=need)
                cur2 = plsc.load_gather(pm_l, [dv])
                return jnp.any(scv > cur2)

            lax.while_loop(lambda carry: carry, bodyw, jnp.bool_(True))
            return 0

        lax.fori_loop(0, C // 16, grp, 0)
        pltpu.sync_copy(scb, score_hbm.at[pl.ds(base, C)])
        return 0

    lax.fori_loop(0, EPT_CHUNKS, chunk, 0)
    pltpu.sync_copy(pm_l, pm_hbm.at[wid])


@functools.partial(
    pl.kernel,
    out_type=[
        jax.ShapeDtypeStruct((EP,), jnp.float32),
        jax.ShapeDtypeStruct((NTILES, NP), jnp.float32),
    ],
    mesh=_mesh(),
    compiler_params=_SC_PARAMS,
    scratch_types=[
        pltpu.VMEM((NP,), jnp.float32),
        pltpu.VMEM((NP,), jnp.float32),
        pltpu.VMEM((EPT,), jnp.float32),
        pltpu.VMEM((EPT,), jnp.int32),
        pltpu.VMEM((EPT,), jnp.float32),
    ],
)
def _s2(score_hbm, dst_hbm, m_hbm, e_hbm, ps_hbm, m_l, s_l, scb, dstb, eb):
    cc = lax.axis_index("c")
    ss = lax.axis_index("s")
    wid = ss * 2 + cc
    base0 = wid * EPT

    pltpu.sync_copy(m_hbm, m_l)

    zero = jnp.zeros((16,), jnp.float32)

    def zz(i, _):
        s_l[pl.ds(i * 16, 16)] = zero
        return 0

    lax.fori_loop(0, NP // 16, zz, 0)

    pltpu.sync_copy(score_hbm.at[pl.ds(base0, EPT)], scb)
    pltpu.sync_copy(dst_hbm.at[pl.ds(base0, EPT)], dstb)

    def grp(i, _):
        dv = dstb[pl.ds(i * 16, 16)]
        sv = scb[pl.ds(i * 16, 16)]
        mv = plsc.load_gather(m_l, [dv])
        ev = jnp.exp(sv - mv)
        eb[pl.ds(i * 16, 16)] = ev
        plsc.addupdate_scatter(s_l, [dv], ev)
        return 0

    lax.fori_loop(0, EPT // 16, grp, 0)

    pltpu.sync_copy(eb, e_hbm.at[pl.ds(base0, EPT)])
    pltpu.sync_copy(s_l, ps_hbm.at[wid])


@functools.partial(
    pl.kernel,
    out_type=jax.ShapeDtypeStruct((2, NP, D), jnp.float32),
    mesh=_mesh(),
    compiler_params=_SC_PARAMS,
    scratch_types=[
        pltpu.VMEM((C,), jnp.int32),
        pltpu.VMEM((C,), jnp.int32),
        pltpu.VMEM((C, D), jnp.float32),
        pltpu.VMEM((C,), jnp.float32),
        pltpu.VMEM((NP,), jnp.float32),
        pltpu.VMEM_SHARED((NP, D), jnp.float32),
        pltpu.SemaphoreType.DMA,
    ],
)
def _s3(e_hbm, s_hbm, src_hbm, dst_hbm, v_hbm, aggp_hbm,
        srcv, dstv, vrows, ab, s_l, agg, sem1):
    cc = lax.axis_index("c")
    ss = lax.axis_index("s")
    wid = ss * 2 + cc
    base0 = wid * EPT
    rows_per_tile = NP // 16

    pltpu.sync_copy(s_hbm, s_l)

    zero = jnp.zeros((16,), jnp.float32)

    def zr(i, _):
        for dd in range(D // 16):
            vrows[i, pl.ds(dd * 16, 16)] = zero
        return 0

    lax.fori_loop(0, C, zr, 0)
    for jj in range(rows_per_tile // C):
        pltpu.sync_copy(vrows, agg.at[pl.ds(ss * rows_per_tile + jj * C, C)])
    plsc.subcore_barrier()

    def chunk(ci, _):
        base = base0 + ci * C
        pltpu.sync_copy(src_hbm.at[pl.ds(base, C)], srcv)
        pltpu.sync_copy(dst_hbm.at[pl.ds(base, C)], dstv)
        pltpu.sync_copy(e_hbm.at[pl.ds(base, C)], ab)
        pltpu.async_copy(v_hbm.at[srcv], vrows, sem1).wait()

        def grp(g, _):
            dv = dstv[pl.ds(g * 16, 16)]
            sv = plsc.load_gather(s_l, [dv])
            av = ab[pl.ds(g * 16, 16)] / sv
            for j in range(16):
                i = g * 16 + j
                a = av[j]
                for dd in range(D // 16):
                    vrows[i, pl.ds(dd * 16, 16)] = (
                        vrows[i, pl.ds(dd * 16, 16)] * a)
            return 0

        lax.fori_loop(0, C // 16, grp, 0)
        pltpu.sync_copy(vrows, agg.at[dstv], add=True)
        return 0

    lax.fori_loop(0, EPT_CHUNKS, chunk, 0)
    plsc.subcore_barrier()
    pltpu.sync_copy(
        agg.at[pl.ds(ss * rows_per_tile, rows_per_tile)],
        aggp_hbm.at[cc, pl.ds(ss * rows_per_tile, rows_per_tile)],
    )


def kernel(x, edge_index, W_proj, b_proj, g1, b1, Wq1, Wk1, Wv1, tau1,
           Wq2, Wk2, Wv2, tau2, Wq3, Wk3, Wv3, tau3, W_ctx, b_ctx, g2, b2):
    f32 = jnp.float32
    i32 = jnp.int32

    src0 = edge_index[0].astype(i32)
    dst0 = edge_index[1].astype(i32)
    loop = jnp.arange(N, dtype=i32)
    n_epad = EP - E - N
    padi = N + (jnp.arange(n_epad, dtype=i32) % (NP - N))
    src = jnp.concatenate([src0, loop, padi])
    dst = jnp.concatenate([dst0, loop, padi])

    xp = jnp.pad(x.astype(f32), ((0, NP - N), (0, 0)))

    def row(a):
        return a.astype(f32).reshape(1, -1)

    h0 = _proj_ln(xp, W_proj.astype(f32), row(b_proj), row(g1), row(b1))

    layers = [
        (Wq1, Wk1, Wv1, tau1),
        (Wq2, Wk2, Wv2, tau2),
        (Wq3, Wk3, Wv3, tau3),
    ]

    hs = [h0]
    aggp = None
    for (wq, wk, wv, tau) in layers:
        tau2d = tau.astype(f32).reshape(1, 1)
        if aggp is None:
            q, k, v = _qkv1(tau2d, h0, wq.astype(f32), wk.astype(f32),
                            wv.astype(f32))
        else:
            hprev, q, k, v = _qkv2(tau2d, aggp[0], aggp[1], wq.astype(f32),
                                   wk.astype(f32), wv.astype(f32))
            hs.append(hprev)
        score, pm = _s1(q, k, src, dst)
        m = _colreduce(pm, _colmax_body).reshape(NP)
        e, ps = _s2(score, dst, m)
        s = _colreduce(ps, _colsum_body).reshape(NP)
        aggp = _s3(e, s, src, dst, v)

    out = _final(hs[0], hs[1], hs[2], aggp[0], aggp[1], W_ctx.astype(f32),
                 row(b_ctx), row(g2), row(b2))
    return out[:N]


# speedup vs baseline: 12.9152x; 1.5127x over previous
import functools

import jax
import jax.numpy as jnp
from jax import lax
from jax.experimental import pallas as pl
from jax.experimental.pallas import tpu as pltpu
from jax.experimental.pallas import tpu_sc as plsc

N = 10000
D = 128
H = 128
OUT = 768
E = 320000

NP = 10240
NTILES = 32
C = 128
NCH = 82
EPT = NCH * C
EP = NTILES * EPT
LN_EPS = 1e-5

_SC_PARAMS = pltpu.CompilerParams(needs_layout_passes=False)


def _mesh():
    return plsc.VectorSubcoreMesh(
        core_axis_name="c", subcore_axis_name="s", num_cores=2, num_subcores=16
    )


def _gelu(x):
    return 0.5 * x * (1.0 + lax.erf(x * (2.0 ** -0.5)))


def _proj_ln_body(x_ref, w_ref, b_ref, g_ref, bb_ref, o_ref):
    y = jnp.dot(x_ref[...], w_ref[...], preferred_element_type=jnp.float32)
    y = y + b_ref[...]
    mu = jnp.mean(y, axis=-1, keepdims=True)
    var = jnp.mean(jnp.square(y - mu), axis=-1, keepdims=True)
    o_ref[...] = (y - mu) * lax.rsqrt(var + LN_EPS) * g_ref[...] + bb_ref[...]


def _proj_ln(x, w, b, g, bb):
    BN = 1024
    return pl.pallas_call(
        _proj_ln_body,
        grid=(NP // BN,),
        in_specs=[
            pl.BlockSpec((BN, D), lambda i: (i, 0)),
            pl.BlockSpec((D, H), lambda i: (0, 0)),
            pl.BlockSpec((1, H), lambda i: (0, 0)),
            pl.BlockSpec((1, H), lambda i: (0, 0)),
            pl.BlockSpec((1, H), lambda i: (0, 0)),
        ],
        out_specs=pl.BlockSpec((BN, H), lambda i: (i, 0)),
        out_shape=jax.ShapeDtypeStruct((NP, H), jnp.float32),
    )(x, w, b, g, bb)


def _qkv1_body(tau_ref, h_ref, wq_ref, wk_ref, wv_ref, q_ref, k_ref, v_ref):
    h = h_ref[...]
    scale = 1.0 / jnp.maximum(tau_ref[0, 0], 0.001)
    q_ref[...] = jnp.dot(h, wq_ref[...], preferred_element_type=jnp.float32) * scale
    k_ref[...] = jnp.dot(h, wk_ref[...], preferred_element_type=jnp.float32)
    v_ref[...] = jnp.dot(h, wv_ref[...], preferred_element_type=jnp.float32)


def _qkv1(tau, h, wq, wk, wv):
    BN = 1024
    return pl.pallas_call(
        _qkv1_body,
        grid=(NP // BN,),
        in_specs=[
            pl.BlockSpec(memory_space=pltpu.SMEM),
            pl.BlockSpec((BN, H), lambda i: (i, 0)),
            pl.BlockSpec((H, H), lambda i: (0, 0)),
            pl.BlockSpec((H, H), lambda i: (0, 0)),
            pl.BlockSpec((H, H), lambda i: (0, 0)),
        ],
        out_specs=[
            pl.BlockSpec((BN, H), lambda i: (i, 0)),
            pl.BlockSpec((BN, H), lambda i: (i, 0)),
            pl.BlockSpec((BN, H), lambda i: (i, 0)),
        ],
        out_shape=[
            jax.ShapeDtypeStruct((NP, H), jnp.float32),
            jax.ShapeDtypeStruct((NP, H), jnp.float32),
            jax.ShapeDtypeStruct((NP, H), jnp.float32),
        ],
    )(tau, h, wq, wk, wv)


def _qkv2_body(tau_ref, a0_ref, a1_ref, wq_ref, wk_ref, wv_ref,
               h_ref, q_ref, k_ref, v_ref):
    h = _gelu(a0_ref[...] + a1_ref[...])
    h_ref[...] = h
    scale = 1.0 / jnp.maximum(tau_ref[0, 0], 0.001)
    q_ref[...] = jnp.dot(h, wq_ref[...], preferred_element_type=jnp.float32) * scale
    k_ref[...] = jnp.dot(h, wk_ref[...], preferred_element_type=jnp.float32)
    v_ref[...] = jnp.dot(h, wv_ref[...], preferred_element_type=jnp.float32)


def _qkv2(tau, a0, a1, wq, wk, wv):
    BN = 1024
    return pl.pallas_call(
        _qkv2_body,
        grid=(NP // BN,),
        in_specs=[
            pl.BlockSpec(memory_space=pltpu.SMEM),
            pl.BlockSpec((BN, H), lambda i: (i, 0)),
            pl.BlockSpec((BN, H), lambda i: (i, 0)),
            pl.BlockSpec((H, H), lambda i: (0, 0)),
            pl.BlockSpec((H, H), lambda i: (0, 0)),
            pl.BlockSpec((H, H), lambda i: (0, 0)),
        ],
        out_specs=[
            pl.BlockSpec((BN, H), lambda i: (i, 0)),
            pl.BlockSpec((BN, H), lambda i: (i, 0)),
            pl.BlockSpec((BN, H), lambda i: (i, 0)),
            pl.BlockSpec((BN, H), lambda i: (i, 0)),
        ],
        out_shape=[
            jax.ShapeDtypeStruct((NP, H), jnp.float32),
            jax.ShapeDtypeStruct((NP, H), jnp.float32),
            jax.ShapeDtypeStruct((NP, H), jnp.float32),
            jax.ShapeDtypeStruct((NP, H), jnp.float32),
        ],
    )(tau, a0, a1, wq, wk, wv)


def _colmax_body(p_ref, o_ref):
    o_ref[...] = jnp.max(p_ref[...], axis=0, keepdims=True)


def _colsum_body(p_ref, o_ref):
    o_ref[...] = jnp.sum(p_ref[...], axis=0, keepdims=True)


def _colreduce(p, body):
    BC = 1280
    return pl.pallas_call(
        body,
        grid=(NP // BC,),
        in_specs=[pl.BlockSpec((NTILES, BC), lambda i: (0, i))],
        out_specs=pl.BlockSpec((1, BC), lambda i: (0, i)),
        out_shape=jax.ShapeDtypeStruct((1, NP), jnp.float32),
    )(p)


def _final_body(h0_ref, h1_ref, h2_ref, a0_ref, a1_ref, w_ref, b_ref,
                g_ref, bb_ref, o_ref):
    h3 = _gelu(a0_ref[...] + a1_ref[...])
    w = w_ref[...]
    acc = jnp.dot(h0_ref[...], w[0:H], preferred_element_type=jnp.float32)
    acc = acc + jnp.dot(h1_ref[...], w[H:2 * H], preferred_element_type=jnp.float32)
    acc = acc + jnp.dot(h2_ref[...], w[2 * H:3 * H], preferred_element_type=jnp.float32)
    acc = acc + jnp.dot(h3, w[3 * H:4 * H], preferred_element_type=jnp.float32)
    y = _gelu(acc + b_ref[...])
    mu = jnp.mean(y, axis=-1, keepdims=True)
    var = jnp.mean(jnp.square(y - mu), axis=-1, keepdims=True)
    o_ref[...] = (y - mu) * lax.rsqrt(var + LN_EPS) * g_ref[...] + bb_ref[...]


def _final(h0, h1, h2, a0, a1, w, b, g, bb):
    BN = 512
    return pl.pallas_call(
        _final_body,
        grid=(NP // BN,),
        in_specs=[
            pl.BlockSpec((BN, H), lambda i: (i, 0)),
            pl.BlockSpec((BN, H), lambda i: (i, 0)),
            pl.BlockSpec((BN, H), lambda i: (i, 0)),
            pl.BlockSpec((BN, H), lambda i: (i, 0)),
            pl.BlockSpec((BN, H), lambda i: (i, 0)),
            pl.BlockSpec((4 * H, OUT), lambda i: (0, 0)),
            pl.BlockSpec((1, OUT), lambda i: (0, 0)),
            pl.BlockSpec((1, OUT), lambda i: (0, 0)),
            pl.BlockSpec((1, OUT), lambda i: (0, 0)),
        ],
        out_specs=pl.BlockSpec((BN, OUT), lambda i: (i, 0)),
        out_shape=jax.ShapeDtypeStruct((NP, OUT), jnp.float32),
    )(h0, h1, h2, a0, a1, w, b, g, bb)


@functools.partial(
    pl.kernel,
    out_type=[
        jax.ShapeDtypeStruct((EP,), jnp.float32),
        jax.ShapeDtypeStruct((NTILES, NP), jnp.float32),
    ],
    mesh=_mesh(),
    compiler_params=_SC_PARAMS,
    scratch_types=[
        pltpu.VMEM((2, C), jnp.int32),
        pltpu.VMEM((2, C), jnp.int32),
        pltpu.VMEM((C, D), jnp.float32),
        pltpu.VMEM((C, D), jnp.float32),
        pltpu.VMEM((C, D), jnp.float32),
        pltpu.VMEM((C, D), jnp.float32),
        pltpu.VMEM((C,), jnp.float32),
        pltpu.VMEM((C,), jnp.float32),
        pltpu.VMEM((NP,), jnp.float32),
        pltpu.SemaphoreType.DMA,
        pltpu.SemaphoreType.DMA,
        pltpu.SemaphoreType.DMA,
        pltpu.SemaphoreType.DMA,
        pltpu.SemaphoreType.DMA,
        pltpu.SemaphoreType.DMA,
    ],
)
def _s1(q_hbm, k_hbm, sd_hbm, score_hbm, pm_hbm,
        sd0, sd1, qr0, kr0, qr1, kr1, scb0, scb1, pm_l,
        gq0, gk0, gq1, gk1, st0, st1):
    cc = lax.axis_index("c")
    ss = lax.axis_index("s")
    wid = ss * 2 + cc
    base0 = wid * EPT
    lanes = lax.iota(jnp.int32, 16)

    neg = jnp.full((16,), -3.0e38, jnp.float32)

    def initb(i, _):
        pm_l[pl.ds(i * 16, 16)] = neg
        return 0

    lax.fori_loop(0, NP // 16, initb, 0)

    bufs = ((sd0, qr0, kr0, scb0, gq0, gk0, st0),
            (sd1, qr1, kr1, scb1, gq1, gk1, st1))

    pltpu.sync_copy(sd_hbm.at[wid * NCH], sd0)
    pltpu.async_copy(q_hbm.at[sd0.at[1]], qr0, gq0)
    pltpu.async_copy(k_hbm.at[sd0.at[0]], kr0, gk0)

    def outer(g2, _):
        for b in range(2):
            sdb, qrows, krows, scb, gq, gk, st = bufs[b]
            nsdb, nqrows, nkrows, _, ngq, ngk, _ = bufs[1 - b]
            ci = g2 * 2 + b
            base = base0 + ci * C

            @pl.when(ci + 1 < NCH)
            def _():
                pltpu.sync_copy(sd_hbm.at[wid * NCH + ci + 1], nsdb)
                pltpu.async_copy(q_hbm.at[nsdb.at[1]], nqrows, ngq)
                pltpu.async_copy(k_hbm.at[nsdb.at[0]], nkrows, ngk)

            pltpu.make_async_copy(q_hbm.at[sdb.at[1]], qrows, gq).wait()
            pltpu.make_async_copy(k_hbm.at[sdb.at[0]], krows, gk).wait()

            @pl.when(ci >= 2)
            def _():
                pltpu.make_async_copy(
                    scb, score_hbm.at[pl.ds(base0, C)], st).wait()

            def grp(g, _):
                scv = jnp.zeros((16,), jnp.float32)
                for j in range(16):
                    i = g * 16 + j
                    a = qrows[i, pl.ds(0, 16)] * krows[i, pl.ds(0, 16)]
                    for dd in range(1, D // 16):
                        a = a + (qrows[i, pl.ds(dd * 16, 16)]
                                 * krows[i, pl.ds(dd * 16, 16)])
                    scv = jnp.where(lanes == j, jnp.sum(a), scv)
                scb[pl.ds(g * 16, 16)] = scv
                dv = sdb[1, pl.ds(g * 16, 16)]

                def bodyw(_carry):
                    cur = plsc.load_gather(pm_l, [dv])
                    need = scv > cur
                    plsc.store_scatter(pm_l, [dv], scv, mask=need)
                    cur2 = plsc.load_gather(pm_l, [dv])
                    return jnp.any(scv > cur2)

                lax.while_loop(lambda carry: carry, bodyw, jnp.bool_(True))
                return 0

            lax.fori_loop(0, C // 16, grp, 0)
            pltpu.async_copy(scb, score_hbm.at[pl.ds(base, C)], st)
        return 0

    lax.fori_loop(0, NCH // 2, outer, 0)
    for b in range(2):
        sdb, qrows, krows, scb, gq, gk, st = bufs[b]
        pltpu.make_async_copy(scb, score_hbm.at[pl.ds(base0, C)], st).wait()
    pltpu.sync_copy(pm_l, pm_hbm.at[wid])


@functools.partial(
    pl.kernel,
    out_type=[
        jax.ShapeDtypeStruct((EP,), jnp.float32),
        jax.ShapeDtypeStruct((NTILES, NP), jnp.float32),
    ],
    mesh=_mesh(),
    compiler_params=_SC_PARAMS,
    scratch_types=[
        pltpu.VMEM((NP,), jnp.float32),
        pltpu.VMEM((NP,), jnp.float32),
        pltpu.VMEM((EPT,), jnp.float32),
        pltpu.VMEM((EPT,), jnp.int32),
        pltpu.VMEM((EPT,), jnp.float32),
    ],
)
def _s2(score_hbm, dst_hbm, m_hbm, e_hbm, ps_hbm, m_l, s_l, scb, dstb, eb):
    cc = lax.axis_index("c")
    ss = lax.axis_index("s")
    wid = ss * 2 + cc
    base0 = wid * EPT

    pltpu.sync_copy(m_hbm, m_l)

    zero = jnp.zeros((16,), jnp.float32)

    def zz(i, _):
        s_l[pl.ds(i * 16, 16)] = zero
        return 0

    lax.fori_loop(0, NP // 16, zz, 0)

    pltpu.sync_copy(score_hbm.at[pl.ds(base0, EPT)], scb)
    pltpu.sync_copy(dst_hbm.at[pl.ds(base0, EPT)], dstb)

    def grp(i, _):
        dv = dstb[pl.ds(i * 16, 16)]
        sv = scb[pl.ds(i * 16, 16)]
        mv = plsc.load_gather(m_l, [dv])
        ev = jnp.exp(sv - mv)
        eb[pl.ds(i * 16, 16)] = ev
        plsc.addupdate_scatter(s_l, [dv], ev)
        return 0

    lax.fori_loop(0, EPT // 16, grp, 0)

    pltpu.sync_copy(eb, e_hbm.at[pl.ds(base0, EPT)])
    pltpu.sync_copy(s_l, ps_hbm.at[wid])


@functools.partial(
    pl.kernel,
    out_type=jax.ShapeDtypeStruct((2, NP, D), jnp.float32),
    mesh=_mesh(),
    compiler_params=_SC_PARAMS,
    scratch_types=[
        pltpu.VMEM((2, C), jnp.int32),
        pltpu.VMEM((2, C), jnp.int32),
        pltpu.VMEM((C, D), jnp.float32),
        pltpu.VMEM((C, D), jnp.float32),
        pltpu.VMEM((C,), jnp.float32),
        pltpu.VMEM((C,), jnp.float32),
        pltpu.VMEM((NP,), jnp.float32),
        pltpu.VMEM_SHARED((NP, D), jnp.float32),
        pltpu.SemaphoreType.DMA,
        pltpu.SemaphoreType.DMA,
        pltpu.SemaphoreType.DMA,
        pltpu.SemaphoreType.DMA,
    ],
)
def _s3(e_hbm, s_hbm, sd_hbm, v_hbm, aggp_hbm,
        sd0, sd1, vr0, vr1, ab0, ab1, s_l, agg, gv0, gv1, sc0, sc1):
    cc = lax.axis_index("c")
    ss = lax.axis_index("s")
    wid = ss * 2 + cc
    base0 = wid * EPT
    rows_per_tile = NP // 16

    pltpu.sync_copy(s_hbm, s_l)

    zero = jnp.zeros((16,), jnp.float32)

    def zr(i, _):
        for dd in range(D // 16):
            vr0[i, pl.ds(dd * 16, 16)] = zero
        return 0

    lax.fori_loop(0, C, zr, 0)
    for jj in range(rows_per_tile // C):
        pltpu.sync_copy(vr0, agg.at[pl.ds(ss * rows_per_tile + jj * C, C)])
    plsc.subcore_barrier()

    bufs = ((sd0, vr0, ab0, gv0, sc0), (sd1, vr1, ab1, gv1, sc1))

    pltpu.sync_copy(sd_hbm.at[wid * NCH], sd0)
    pltpu.sync_copy(e_hbm.at[pl.ds(base0, C)], ab0)
    pltpu.async_copy(v_hbm.at[sd0.at[0]], vr0, gv0)

    def outer(g2, _):
        for b in range(2):
            sdb, vrows, ab, gv, sc = bufs[b]
            nsdb, nvrows, nab, ngv, nsc = bufs[1 - b]
            ci = g2 * 2 + b
            base = base0 + ci * C

            @pl.when((ci + 1 < NCH) & (ci >= 1))
            def _():
                pltpu.make_async_copy(nvrows, agg.at[nsdb.at[1]], nsc).wait()

            @pl.when(ci + 1 < NCH)
            def _():
                nbase = base0 + (ci + 1) * C
                pltpu.sync_copy(sd_hbm.at[wid * NCH + ci + 1], nsdb)
                pltpu.sync_copy(e_hbm.at[pl.ds(nbase, C)], nab)
                pltpu.async_copy(v_hbm.at[nsdb.at[0]], nvrows, ngv)

            pltpu.make_async_copy(v_hbm.at[sdb.at[0]], vrows, gv).wait()

            def grp(g, _):
                dv = sdb[1, pl.ds(g * 16, 16)]
                sv = plsc.load_gather(s_l, [dv])
                av = ab[pl.ds(g * 16, 16)] / sv
                for j in range(16):
                    i = g * 16 + j
                    a = av[j]
                    for dd in range(D // 16):
                        vrows[i, pl.ds(dd * 16, 16)] = (
                            vrows[i, pl.ds(dd * 16, 16)] * a)
                return 0

            lax.fori_loop(0, C // 16, grp, 0)
            pltpu.async_copy(vrows, agg.at[sdb.at[1]], sc, add=True)
        return 0

    lax.fori_loop(0, NCH // 2, outer, 0)
    for b in range(2):
        sdb, vrows, ab, gv, sc = bufs[b]
        pltpu.make_async_copy(vrows, agg.at[sdb.at[1]], sc).wait()
    plsc.subcore_barrier()
    pltpu.sync_copy(
        agg.at[pl.ds(ss * rows_per_tile, rows_per_tile)],
        aggp_hbm.at[cc, pl.ds(ss * rows_per_tile, rows_per_tile)],
    )


def kernel(x, edge_index, W_proj, b_proj, g1, b1, Wq1, Wk1, Wv1, tau1,
           Wq2, Wk2, Wv2, tau2, Wq3, Wk3, Wv3, tau3, W_ctx, b_ctx, g2, b2):
    f32 = jnp.float32
    i32 = jnp.int32

    src0 = edge_index[0].astype(i32)
    dst0 = edge_index[1].astype(i32)
    loop = jnp.arange(N, dtype=i32)
    n_epad = EP - E - N
    padi = N + (jnp.arange(n_epad, dtype=i32) % (NP - N))
    src = jnp.concatenate([src0, loop, padi])
    dst = jnp.concatenate([dst0, loop, padi])
    sd = jnp.stack(
        [src.reshape(NTILES * NCH, C), dst.reshape(NTILES * NCH, C)], axis=1)

    xp = jnp.pad(x.astype(f32), ((0, NP - N), (0, 0)))

    def row(a):
        return a.astype(f32).reshape(1, -1)

    h0 = _proj_ln(xp, W_proj.astype(f32), row(b_proj), row(g1), row(b1))

    layers = [
        (Wq1, Wk1, Wv1, tau1),
        (Wq2, Wk2, Wv2, tau2),
        (Wq3, Wk3, Wv3, tau3),
    ]

    hs = [h0]
    aggp = None
    for (wq, wk, wv, tau) in layers:
        tau2d = tau.astype(f32).reshape(1, 1)
        if aggp is None:
            q, k, v = _qkv1(tau2d, h0, wq.astype(f32), wk.astype(f32),
                            wv.astype(f32))
        else:
            hprev, q, k, v = _qkv2(tau2d, aggp[0], aggp[1], wq.astype(f32),
                                   wk.astype(f32), wv.astype(f32))
            hs.append(hprev)
        score, pm = _s1(q, k, sd)
        m = _colreduce(pm, _colmax_body).reshape(NP)
        e, ps = _s2(score, dst, m)
        s = _colreduce(ps, _colsum_body).reshape(NP)
        aggp = _s3(e, s, sd, v)

    out = _final(hs[0], hs[1], hs[2], aggp[0], aggp[1], W_ctx.astype(f32),
                 row(b_ctx), row(g2), row(b2))
    return out[:N]


# speedup vs baseline: 16.6622x; 1.2901x over previous
import functools

import jax
import jax.numpy as jnp
from jax import lax
from jax.experimental import pallas as pl
from jax.experimental.pallas import tpu as pltpu
from jax.experimental.pallas import tpu_sc as plsc

N = 10000
D = 128
H = 128
OUT = 768
E = 320000

NP = 10240
NTILES = 32
C = 128
NCH = 82
EPT = NCH * C
EP = NTILES * EPT
LN_EPS = 1e-5

_SC_PARAMS = pltpu.CompilerParams(needs_layout_passes=False)


def _mesh():
    return plsc.VectorSubcoreMesh(
        core_axis_name="c", subcore_axis_name="s", num_cores=2, num_subcores=16
    )


def _gelu(x):
    return 0.5 * x * (1.0 + lax.erf(x * (2.0 ** -0.5)))


def _proj_ln_body(x_ref, w_ref, b_ref, g_ref, bb_ref, o_ref):
    y = jnp.dot(x_ref[...], w_ref[...], preferred_element_type=jnp.float32)
    y = y + b_ref[...]
    mu = jnp.mean(y, axis=-1, keepdims=True)
    var = jnp.mean(jnp.square(y - mu), axis=-1, keepdims=True)
    o_ref[...] = (y - mu) * lax.rsqrt(var + LN_EPS) * g_ref[...] + bb_ref[...]


def _proj_ln(x, w, b, g, bb):
    BN = 1024
    return pl.pallas_call(
        _proj_ln_body,
        grid=(NP // BN,),
        in_specs=[
            pl.BlockSpec((BN, D), lambda i: (i, 0)),
            pl.BlockSpec((D, H), lambda i: (0, 0)),
            pl.BlockSpec((1, H), lambda i: (0, 0)),
            pl.BlockSpec((1, H), lambda i: (0, 0)),
            pl.BlockSpec((1, H), lambda i: (0, 0)),
        ],
        out_specs=pl.BlockSpec((BN, H), lambda i: (i, 0)),
        out_shape=jax.ShapeDtypeStruct((NP, H), jnp.float32),
    )(x, w, b, g, bb)


def _qkv1_body(tau_ref, h_ref, wq_ref, wk_ref, wv_ref, q_ref, k_ref, v_ref):
    h = h_ref[...]
    scale = 1.0 / jnp.maximum(tau_ref[0, 0], 0.001)
    q_ref[...] = jnp.dot(h, wq_ref[...], preferred_element_type=jnp.float32) * scale
    k_ref[...] = jnp.dot(h, wk_ref[...], preferred_element_type=jnp.float32)
    v_ref[...] = jnp.dot(h, wv_ref[...], preferred_element_type=jnp.float32)


def _qkv1(tau, h, wq, wk, wv):
    BN = 1024
    return pl.pallas_call(
        _qkv1_body,
        grid=(NP // BN,),
        in_specs=[
            pl.BlockSpec(memory_space=pltpu.SMEM),
            pl.BlockSpec((BN, H), lambda i: (i, 0)),
            pl.BlockSpec((H, H), lambda i: (0, 0)),
            pl.BlockSpec((H, H), lambda i: (0, 0)),
            pl.BlockSpec((H, H), lambda i: (0, 0)),
        ],
        out_specs=[
            pl.BlockSpec((BN, H), lambda i: (i, 0)),
            pl.BlockSpec((BN, H), lambda i: (i, 0)),
            pl.BlockSpec((BN, H), lambda i: (i, 0)),
        ],
        out_shape=[
            jax.ShapeDtypeStruct((NP, H), jnp.float32),
            jax.ShapeDtypeStruct((NP, H), jnp.float32),
            jax.ShapeDtypeStruct((NP, H), jnp.float32),
        ],
    )(tau, h, wq, wk, wv)


def _qkv2_body(tau_ref, a0_ref, a1_ref, wq_ref, wk_ref, wv_ref,
               h_ref, q_ref, k_ref, v_ref):
    h = _gelu(a0_ref[...] + a1_ref[...])
    h_ref[...] = h
    scale = 1.0 / jnp.maximum(tau_ref[0, 0], 0.001)
    q_ref[...] = jnp.dot(h, wq_ref[...], preferred_element_type=jnp.float32) * scale
    k_ref[...] = jnp.dot(h, wk_ref[...], preferred_element_type=jnp.float32)
    v_ref[...] = jnp.dot(h, wv_ref[...], preferred_element_type=jnp.float32)


def _qkv2(tau, a0, a1, wq, wk, wv):
    BN = 1024
    return pl.pallas_call(
        _qkv2_body,
        grid=(NP // BN,),
        in_specs=[
            pl.BlockSpec(memory_space=pltpu.SMEM),
            pl.BlockSpec((BN, H), lambda i: (i, 0)),
            pl.BlockSpec((BN, H), lambda i: (i, 0)),
            pl.BlockSpec((H, H), lambda i: (0, 0)),
            pl.BlockSpec((H, H), lambda i: (0, 0)),
            pl.BlockSpec((H, H), lambda i: (0, 0)),
        ],
        out_specs=[
            pl.BlockSpec((BN, H), lambda i: (i, 0)),
            pl.BlockSpec((BN, H), lambda i: (i, 0)),
            pl.BlockSpec((BN, H), lambda i: (i, 0)),
            pl.BlockSpec((BN, H), lambda i: (i, 0)),
        ],
        out_shape=[
            jax.ShapeDtypeStruct((NP, H), jnp.float32),
            jax.ShapeDtypeStruct((NP, H), jnp.float32),
            jax.ShapeDtypeStruct((NP, H), jnp.float32),
            jax.ShapeDtypeStruct((NP, H), jnp.float32),
        ],
    )(tau, a0, a1, wq, wk, wv)


def _colmax_body(p_ref, o_ref):
    o_ref[...] = jnp.max(p_ref[...], axis=0, keepdims=True)


def _colsum_body(p_ref, o_ref):
    o_ref[...] = jnp.sum(p_ref[...], axis=0, keepdims=True)


def _colreduce(p, body):
    BC = 1280
    return pl.pallas_call(
        body,
        grid=(NP // BC,),
        in_specs=[pl.BlockSpec((NTILES, BC), lambda i: (0, i))],
        out_specs=pl.BlockSpec((1, BC), lambda i: (0, i)),
        out_shape=jax.ShapeDtypeStruct((1, NP), jnp.float32),
    )(p)


def _final_body(h0_ref, h1_ref, h2_ref, a0_ref, a1_ref, w_ref, b_ref,
                g_ref, bb_ref, o_ref):
    h3 = _gelu(a0_ref[...] + a1_ref[...])
    w = w_ref[...]
    acc = jnp.dot(h0_ref[...], w[0:H], preferred_element_type=jnp.float32)
    acc = acc + jnp.dot(h1_ref[...], w[H:2 * H], preferred_element_type=jnp.float32)
    acc = acc + jnp.dot(h2_ref[...], w[2 * H:3 * H], preferred_element_type=jnp.float32)
    acc = acc + jnp.dot(h3, w[3 * H:4 * H], preferred_element_type=jnp.float32)
    y = _gelu(acc + b_ref[...])
    mu = jnp.mean(y, axis=-1, keepdims=True)
    var = jnp.mean(jnp.square(y - mu), axis=-1, keepdims=True)
    o_ref[...] = (y - mu) * lax.rsqrt(var + LN_EPS) * g_ref[...] + bb_ref[...]


def _final(h0, h1, h2, a0, a1, w, b, g, bb):
    BN = 512
    return pl.pallas_call(
        _final_body,
        grid=(NP // BN,),
        in_specs=[
            pl.BlockSpec((BN, H), lambda i: (i, 0)),
            pl.BlockSpec((BN, H), lambda i: (i, 0)),
            pl.BlockSpec((BN, H), lambda i: (i, 0)),
            pl.BlockSpec((BN, H), lambda i: (i, 0)),
            pl.BlockSpec((BN, H), lambda i: (i, 0)),
            pl.BlockSpec((4 * H, OUT), lambda i: (0, 0)),
            pl.BlockSpec((1, OUT), lambda i: (0, 0)),
            pl.BlockSpec((1, OUT), lambda i: (0, 0)),
            pl.BlockSpec((1, OUT), lambda i: (0, 0)),
        ],
        out_specs=pl.BlockSpec((BN, OUT), lambda i: (i, 0)),
        out_shape=jax.ShapeDtypeStruct((NP, OUT), jnp.float32),
    )(h0, h1, h2, a0, a1, w, b, g, bb)


@functools.partial(
    pl.kernel,
    out_type=[
        jax.ShapeDtypeStruct((EP,), jnp.float32),
        jax.ShapeDtypeStruct((NTILES, NP), jnp.float32),
    ],
    mesh=_mesh(),
    compiler_params=_SC_PARAMS,
    scratch_types=[
        pltpu.VMEM((2, C), jnp.int32),
        pltpu.VMEM((2, C), jnp.int32),
        pltpu.VMEM((C, D), jnp.float32),
        pltpu.VMEM((C, D), jnp.float32),
        pltpu.VMEM((C, D), jnp.float32),
        pltpu.VMEM((C, D), jnp.float32),
        pltpu.VMEM((C,), jnp.float32),
        pltpu.VMEM((C,), jnp.float32),
        pltpu.VMEM((NP,), jnp.float32),
        pltpu.SemaphoreType.DMA,
        pltpu.SemaphoreType.DMA,
        pltpu.SemaphoreType.DMA,
        pltpu.SemaphoreType.DMA,
        pltpu.SemaphoreType.DMA,
        pltpu.SemaphoreType.DMA,
    ],
)
def _s1(q_hbm, k_hbm, sd_hbm, score_hbm, pm_hbm,
        sd0, sd1, qr0, kr0, qr1, kr1, scb0, scb1, pm_l,
        gq0, gk0, gq1, gk1, st0, st1):
    cc = lax.axis_index("c")
    ss = lax.axis_index("s")
    wid = ss * 2 + cc
    base0 = wid * EPT
    lanes = lax.iota(jnp.int32, 16)

    neg = jnp.full((16,), -3.0e38, jnp.float32)

    def initb(i, _):
        pm_l[pl.ds(i * 16, 16)] = neg
        return 0

    lax.fori_loop(0, NP // 16, initb, 0)

    bufs = ((sd0, qr0, kr0, scb0, gq0, gk0, st0),
            (sd1, qr1, kr1, scb1, gq1, gk1, st1))

    pltpu.sync_copy(sd_hbm.at[wid * NCH], sd0)
    pltpu.async_copy(q_hbm.at[sd0.at[1]], qr0, gq0)
    pltpu.async_copy(k_hbm.at[sd0.at[0]], kr0, gk0)

    def outer(g2, _):
        for b in range(2):
            sdb, qrows, krows, scb, gq, gk, st = bufs[b]
            nsdb, nqrows, nkrows, _, ngq, ngk, _ = bufs[1 - b]
            ci = g2 * 2 + b
            base = base0 + ci * C

            @pl.when(ci + 1 < NCH)
            def _():
                pltpu.sync_copy(sd_hbm.at[wid * NCH + ci + 1], nsdb)
                pltpu.async_copy(q_hbm.at[nsdb.at[1]], nqrows, ngq)
                pltpu.async_copy(k_hbm.at[nsdb.at[0]], nkrows, ngk)

            pltpu.make_async_copy(q_hbm.at[sdb.at[1]], qrows, gq).wait()
            pltpu.make_async_copy(k_hbm.at[sdb.at[0]], krows, gk).wait()

            @pl.when(ci >= 2)
            def _():
                pltpu.make_async_copy(
                    scb, score_hbm.at[pl.ds(base0, C)], st).wait()

            def grp(g, _):
                def dotj(j, scv):
                    i = g * 16 + j
                    a = qrows[i, pl.ds(0, 16)] * krows[i, pl.ds(0, 16)]
                    for dd in range(1, D // 16):
                        a = a + (qrows[i, pl.ds(dd * 16, 16)]
                                 * krows[i, pl.ds(dd * 16, 16)])
                    return jnp.where(lanes == j, jnp.sum(a), scv)

                scv = lax.fori_loop(0, 16, dotj, jnp.zeros((16,), jnp.float32),
                                    unroll=4)
                scb[pl.ds(g * 16, 16)] = scv
                dv = sdb[1, pl.ds(g * 16, 16)]

                def bodyw(_carry):
                    cur = plsc.load_gather(pm_l, [dv])
                    need = scv > cur
                    plsc.store_scatter(pm_l, [dv], scv, mask=need)
                    cur2 = plsc.load_gather(pm_l, [dv])
                    return jnp.any(scv > cur2)

                lax.while_loop(lambda carry: carry, bodyw, jnp.bool_(True))
                return 0

            lax.fori_loop(0, C // 16, grp, 0)
            pltpu.async_copy(scb, score_hbm.at[pl.ds(base, C)], st)
        return 0

    lax.fori_loop(0, NCH // 2, outer, 0)
    for b in range(2):
        sdb, qrows, krows, scb, gq, gk, st = bufs[b]
        pltpu.make_async_copy(scb, score_hbm.at[pl.ds(base0, C)], st).wait()
    pltpu.sync_copy(pm_l, pm_hbm.at[wid])


@functools.partial(
    pl.kernel,
    out_type=[
        jax.ShapeDtypeStruct((EP,), jnp.float32),
        jax.ShapeDtypeStruct((NTILES, NP), jnp.float32),
    ],
    mesh=_mesh(),
    compiler_params=_SC_PARAMS,
    scratch_types=[
        pltpu.VMEM((NP,), jnp.float32),
        pltpu.VMEM((NP,), jnp.float32),
        pltpu.VMEM((EPT,), jnp.float32),
        pltpu.VMEM((EPT,), jnp.int32),
        pltpu.VMEM((EPT,), jnp.float32),
    ],
)
def _s2(score_hbm, dst_hbm, m_hbm, e_hbm, ps_hbm, m_l, s_l, scb, dstb, eb):
    cc = lax.axis_index("c")
    ss = lax.axis_index("s")
    wid = ss * 2 + cc
    base0 = wid * EPT

    pltpu.sync_copy(m_hbm, m_l)

    zero = jnp.zeros((16,), jnp.float32)

    def zz(i, _):
        s_l[pl.ds(i * 16, 16)] = zero
        return 0

    lax.fori_loop(0, NP // 16, zz, 0)

    pltpu.sync_copy(score_hbm.at[pl.ds(base0, EPT)], scb)
    pltpu.sync_copy(dst_hbm.at[pl.ds(base0, EPT)], dstb)

    def grp(i, _):
        dv = dstb[pl.ds(i * 16, 16)]
        sv = scb[pl.ds(i * 16, 16)]
        mv = plsc.load_gather(m_l, [dv])
        ev = jnp.exp(sv - mv)
        eb[pl.ds(i * 16, 16)] = ev
        plsc.addupdate_scatter(s_l, [dv], ev)
        return 0

    lax.fori_loop(0, EPT // 16, grp, 0)

    pltpu.sync_copy(eb, e_hbm.at[pl.ds(base0, EPT)])
    pltpu.sync_copy(s_l, ps_hbm.at[wid])


@functools.partial(
    pl.kernel,
    out_type=jax.ShapeDtypeStruct((2, NP, D), jnp.float32),
    mesh=_mesh(),
    compiler_params=_SC_PARAMS,
    scratch_types=[
        pltpu.VMEM((2, C), jnp.int32),
        pltpu.VMEM((2, C), jnp.int32),
        pltpu.VMEM((C, D), jnp.float32),
        pltpu.VMEM((C, D), jnp.float32),
        pltpu.VMEM((C,), jnp.float32),
        pltpu.VMEM((C,), jnp.float32),
        pltpu.VMEM((NP,), jnp.float32),
        pltpu.VMEM_SHARED((NP, D), jnp.float32),
        pltpu.SemaphoreType.DMA,
        pltpu.SemaphoreType.DMA,
        pltpu.SemaphoreType.DMA,
        pltpu.SemaphoreType.DMA,
    ],
)
def _s3(e_hbm, s_hbm, sd_hbm, v_hbm, aggp_hbm,
        sd0, sd1, vr0, vr1, ab0, ab1, s_l, agg, gv0, gv1, sc0, sc1):
    cc = lax.axis_index("c")
    ss = lax.axis_index("s")
    wid = ss * 2 + cc
    base0 = wid * EPT
    rows_per_tile = NP // 16

    pltpu.sync_copy(s_hbm, s_l)

    zero = jnp.zeros((16,), jnp.float32)

    def zr(i, _):
        for dd in range(D // 16):
            vr0[i, pl.ds(dd * 16, 16)] = zero
        return 0

    lax.fori_loop(0, C, zr, 0)
    for jj in range(rows_per_tile // C):
        pltpu.sync_copy(vr0, agg.at[pl.ds(ss * rows_per_tile + jj * C, C)])
    plsc.subcore_barrier()

    bufs = ((sd0, vr0, ab0, gv0, sc0), (sd1, vr1, ab1, gv1, sc1))

    pltpu.sync_copy(sd_hbm.at[wid * NCH], sd0)
    pltpu.sync_copy(e_hbm.at[pl.ds(base0, C)], ab0)
    pltpu.async_copy(v_hbm.at[sd0.at[0]], vr0, gv0)

    def outer(g2, _):
        for b in range(2):
            sdb, vrows, ab, gv, sc = bufs[b]
            nsdb, nvrows, nab, ngv, nsc = bufs[1 - b]
            ci = g2 * 2 + b
            base = base0 + ci * C

            @pl.when((ci + 1 < NCH) & (ci >= 1))
            def _():
                pltpu.make_async_copy(nvrows, agg.at[nsdb.at[1]], nsc).wait()

            @pl.when(ci + 1 < NCH)
            def _():
                nbase = base0 + (ci + 1) * C
                pltpu.sync_copy(sd_hbm.at[wid * NCH + ci + 1], nsdb)
                pltpu.sync_copy(e_hbm.at[pl.ds(nbase, C)], nab)
                pltpu.async_copy(v_hbm.at[nsdb.at[0]], nvrows, ngv)

            pltpu.make_async_copy(v_hbm.at[sdb.at[0]], vrows, gv).wait()

            def grp(g, _):
                dv = sdb[1, pl.ds(g * 16, 16)]
                sv = plsc.load_gather(s_l, [dv])
                av = ab[pl.ds(g * 16, 16)] / sv
                for j in range(16):
                    i = g * 16 + j
                    a = av[j]
                    for dd in range(D // 16):
                        vrows[i, pl.ds(dd * 16, 16)] = (
                            vrows[i, pl.ds(dd * 16, 16)] * a)
                return 0

            lax.fori_loop(0, C // 16, grp, 0)
            pltpu.async_copy(vrows, agg.at[sdb.at[1]], sc, add=True)
        return 0

    lax.fori_loop(0, NCH // 2, outer, 0)
    for b in range(2):
        sdb, vrows, ab, gv, sc = bufs[b]
        pltpu.make_async_copy(vrows, agg.at[sdb.at[1]], sc).wait()
    plsc.subcore_barrier()
    pltpu.sync_copy(
        agg.at[pl.ds(ss * rows_per_tile, rows_per_tile)],
        aggp_hbm.at[cc, pl.ds(ss * rows_per_tile, rows_per_tile)],
    )


def kernel(x, edge_index, W_proj, b_proj, g1, b1, Wq1, Wk1, Wv1, tau1,
           Wq2, Wk2, Wv2, tau2, Wq3, Wk3, Wv3, tau3, W_ctx, b_ctx, g2, b2):
    f32 = jnp.float32
    i32 = jnp.int32

    src0 = edge_index[0].astype(i32)
    dst0 = edge_index[1].astype(i32)
    loop = jnp.arange(N, dtype=i32)
    n_epad = EP - E - N
    padi = N + (jnp.arange(n_epad, dtype=i32) % (NP - N))
    src = jnp.concatenate([src0, loop, padi])
    dst = jnp.concatenate([dst0, loop, padi])
    sd = jnp.stack(
        [src.reshape(NTILES * NCH, C), dst.reshape(NTILES * NCH, C)], axis=1)

    xp = jnp.pad(x.astype(f32), ((0, NP - N), (0, 0)))

    def row(a):
        return a.astype(f32).reshape(1, -1)

    h0 = _proj_ln(xp, W_proj.astype(f32), row(b_proj), row(g1), row(b1))

    layers = [
        (Wq1, Wk1, Wv1, tau1),
        (Wq2, Wk2, Wv2, tau2),
        (Wq3, Wk3, Wv3, tau3),
    ]

    hs = [h0]
    aggp = None
    for (wq, wk, wv, tau) in layers:
        tau2d = tau.astype(f32).reshape(1, 1)
        if aggp is None:
            q, k, v = _qkv1(tau2d, h0, wq.astype(f32), wk.astype(f32),
                            wv.astype(f32))
        else:
            hprev, q, k, v = _qkv2(tau2d, aggp[0], aggp[1], wq.astype(f32),
                                   wk.astype(f32), wv.astype(f32))
            hs.append(hprev)
        score, pm = _s1(q, k, sd)
        m = _colreduce(pm, _colmax_body).reshape(NP)
        e, ps = _s2(score, dst, m)
        s = _colreduce(ps, _colsum_body).reshape(NP)
        aggp = _s3(e, s, sd, v)

    out = _final(hs[0], hs[1], hs[2], aggp[0], aggp[1], W_ctx.astype(f32),
                 row(b_ctx), row(g2), row(b2))
    return out[:N]


# speedup vs baseline: 19.8698x; 1.1925x over previous
import functools

import jax
import jax.numpy as jnp
from jax import lax
from jax.experimental import pallas as pl
from jax.experimental.pallas import tpu as pltpu
from jax.experimental.pallas import tpu_sc as plsc

N = 10000
D = 128
H = 128
OUT = 768
E = 320000

NP = 10240
NTILES = 32
C = 128
NCH = 82
EPT = NCH * C
EP = NTILES * EPT
LN_EPS = 1e-5

_SC_PARAMS = pltpu.CompilerParams(needs_layout_passes=False)


def _mesh():
    return plsc.VectorSubcoreMesh(
        core_axis_name="c", subcore_axis_name="s", num_cores=2, num_subcores=16
    )


def _gelu(x):
    return 0.5 * x * (1.0 + lax.erf(x * (2.0 ** -0.5)))


def _proj_ln_body(x_ref, w_ref, b_ref, g_ref, bb_ref, o_ref):
    y = jnp.dot(x_ref[...], w_ref[...], preferred_element_type=jnp.float32)
    y = y + b_ref[...]
    mu = jnp.mean(y, axis=-1, keepdims=True)
    var = jnp.mean(jnp.square(y - mu), axis=-1, keepdims=True)
    o_ref[...] = (y - mu) * lax.rsqrt(var + LN_EPS) * g_ref[...] + bb_ref[...]


def _proj_ln(x, w, b, g, bb):
    BN = 1024
    return pl.pallas_call(
        _proj_ln_body,
        grid=(NP // BN,),
        in_specs=[
            pl.BlockSpec((BN, D), lambda i: (i, 0)),
            pl.BlockSpec((D, H), lambda i: (0, 0)),
            pl.BlockSpec((1, H), lambda i: (0, 0)),
            pl.BlockSpec((1, H), lambda i: (0, 0)),
            pl.BlockSpec((1, H), lambda i: (0, 0)),
        ],
        out_specs=pl.BlockSpec((BN, H), lambda i: (i, 0)),
        out_shape=jax.ShapeDtypeStruct((NP, H), jnp.float32),
    )(x, w, b, g, bb)


def _qkv1_body(tau_ref, h_ref, wq_ref, wk_ref, wv_ref, q_ref, k_ref, v_ref):
    h = h_ref[...]
    scale = 1.0 / jnp.maximum(tau_ref[0, 0], 0.001)
    q_ref[...] = jnp.dot(h, wq_ref[...], preferred_element_type=jnp.float32) * scale
    k_ref[...] = jnp.dot(h, wk_ref[...], preferred_element_type=jnp.float32)
    v_ref[...] = jnp.dot(h, wv_ref[...], preferred_element_type=jnp.float32)


def _qkv1(tau, h, wq, wk, wv):
    BN = 1024
    return pl.pallas_call(
        _qkv1_body,
        grid=(NP // BN,),
        in_specs=[
            pl.BlockSpec(memory_space=pltpu.SMEM),
            pl.BlockSpec((BN, H), lambda i: (i, 0)),
            pl.BlockSpec((H, H), lambda i: (0, 0)),
            pl.BlockSpec((H, H), lambda i: (0, 0)),
            pl.BlockSpec((H, H), lambda i: (0, 0)),
        ],
        out_specs=[
            pl.BlockSpec((BN, H), lambda i: (i, 0)),
            pl.BlockSpec((BN, H), lambda i: (i, 0)),
            pl.BlockSpec((BN, H), lambda i: (i, 0)),
        ],
        out_shape=[
            jax.ShapeDtypeStruct((NP, H), jnp.float32),
            jax.ShapeDtypeStruct((NP, H), jnp.float32),
            jax.ShapeDtypeStruct((NP, H), jnp.float32),
        ],
    )(tau, h, wq, wk, wv)


def _qkv2_body(tau_ref, a0_ref, a1_ref, wq_ref, wk_ref, wv_ref,
               h_ref, q_ref, k_ref, v_ref):
    h = _gelu(a0_ref[...] + a1_ref[...])
    h_ref[...] = h
    scale = 1.0 / jnp.maximum(tau_ref[0, 0], 0.001)
    q_ref[...] = jnp.dot(h, wq_ref[...], preferred_element_type=jnp.float32) * scale
    k_ref[...] = jnp.dot(h, wk_ref[...], preferred_element_type=jnp.float32)
    v_ref[...] = jnp.dot(h, wv_ref[...], preferred_element_type=jnp.float32)


def _qkv2(tau, a0, a1, wq, wk, wv):
    BN = 1024
    return pl.pallas_call(
        _qkv2_body,
        grid=(NP // BN,),
        in_specs=[
            pl.BlockSpec(memory_space=pltpu.SMEM),
            pl.BlockSpec((BN, H), lambda i: (i, 0)),
            pl.BlockSpec((BN, H), lambda i: (i, 0)),
            pl.BlockSpec((H, H), lambda i: (0, 0)),
            pl.BlockSpec((H, H), lambda i: (0, 0)),
            pl.BlockSpec((H, H), lambda i: (0, 0)),
        ],
        out_specs=[
            pl.BlockSpec((BN, H), lambda i: (i, 0)),
            pl.BlockSpec((BN, H), lambda i: (i, 0)),
            pl.BlockSpec((BN, H), lambda i: (i, 0)),
            pl.BlockSpec((BN, H), lambda i: (i, 0)),
        ],
        out_shape=[
            jax.ShapeDtypeStruct((NP, H), jnp.float32),
            jax.ShapeDtypeStruct((NP, H), jnp.float32),
            jax.ShapeDtypeStruct((NP, H), jnp.float32),
            jax.ShapeDtypeStruct((NP, H), jnp.float32),
        ],
    )(tau, a0, a1, wq, wk, wv)


def _colmax_body(p_ref, o_ref):
    o_ref[...] = jnp.max(p_ref[...], axis=0, keepdims=True)


def _colsum_body(p_ref, o_ref):
    o_ref[...] = jnp.sum(p_ref[...], axis=0, keepdims=True)


def _colreduce(p, body):
    BC = 1280
    return pl.pallas_call(
        body,
        grid=(NP // BC,),
        in_specs=[pl.BlockSpec((NTILES, BC), lambda i: (0, i))],
        out_specs=pl.BlockSpec((1, BC), lambda i: (0, i)),
        out_shape=jax.ShapeDtypeStruct((1, NP), jnp.float32),
    )(p)


def _final_body(h0_ref, h1_ref, h2_ref, a0_ref, a1_ref, w_ref, b_ref,
                g_ref, bb_ref, o_ref):
    h3 = _gelu(a0_ref[...] + a1_ref[...])
    w = w_ref[...]
    acc = jnp.dot(h0_ref[...], w[0:H], preferred_element_type=jnp.float32)
    acc = acc + jnp.dot(h1_ref[...], w[H:2 * H], preferred_element_type=jnp.float32)
    acc = acc + jnp.dot(h2_ref[...], w[2 * H:3 * H], preferred_element_type=jnp.float32)
    acc = acc + jnp.dot(h3, w[3 * H:4 * H], preferred_element_type=jnp.float32)
    y = _gelu(acc + b_ref[...])
    mu = jnp.mean(y, axis=-1, keepdims=True)
    var = jnp.mean(jnp.square(y - mu), axis=-1, keepdims=True)
    o_ref[...] = (y - mu) * lax.rsqrt(var + LN_EPS) * g_ref[...] + bb_ref[...]


def _final(h0, h1, h2, a0, a1, w, b, g, bb):
    BN = 512
    return pl.pallas_call(
        _final_body,
        grid=(NP // BN,),
        in_specs=[
            pl.BlockSpec((BN, H), lambda i: (i, 0)),
            pl.BlockSpec((BN, H), lambda i: (i, 0)),
            pl.BlockSpec((BN, H), lambda i: (i, 0)),
            pl.BlockSpec((BN, H), lambda i: (i, 0)),
            pl.BlockSpec((BN, H), lambda i: (i, 0)),
            pl.BlockSpec((4 * H, OUT), lambda i: (0, 0)),
            pl.BlockSpec((1, OUT), lambda i: (0, 0)),
            pl.BlockSpec((1, OUT), lambda i: (0, 0)),
            pl.BlockSpec((1, OUT), lambda i: (0, 0)),
        ],
        out_specs=pl.BlockSpec((BN, OUT), lambda i: (i, 0)),
        out_shape=jax.ShapeDtypeStruct((NP, OUT), jnp.float32),
    )(h0, h1, h2, a0, a1, w, b, g, bb)


@functools.partial(
    pl.kernel,
    out_type=[
        jax.ShapeDtypeStruct((EP,), jnp.float32),
        jax.ShapeDtypeStruct((NTILES, NP), jnp.float32),
    ],
    mesh=_mesh(),
    compiler_params=_SC_PARAMS,
    scratch_types=[
        pltpu.VMEM((2, C), jnp.int32),
        pltpu.VMEM((2, C), jnp.int32),
        pltpu.VMEM((C, D), jnp.float32),
        pltpu.VMEM((C, D), jnp.float32),
        pltpu.VMEM((C, D), jnp.float32),
        pltpu.VMEM((C, D), jnp.float32),
        pltpu.VMEM((C,), jnp.float32),
        pltpu.VMEM((C,), jnp.float32),
        pltpu.VMEM((C,), jnp.int32),
        pltpu.VMEM((NP,), jnp.float32),
        pltpu.SemaphoreType.DMA,
        pltpu.SemaphoreType.DMA,
        pltpu.SemaphoreType.DMA,
        pltpu.SemaphoreType.DMA,
        pltpu.SemaphoreType.DMA,
        pltpu.SemaphoreType.DMA,
        pltpu.SemaphoreType.DMA,
        pltpu.SemaphoreType.DMA,
    ],
)
def _s1(q_hbm, k_hbm, sd_hbm, score_hbm, pm_hbm,
        sd0, sd1, qr0, kr0, qr1, kr1, scb0, scb1, dstc, pm_l,
        gq0, gk0, gq1, gk1, st0, st1, sdm0, sdm1):
    cc = lax.axis_index("c")
    ss = lax.axis_index("s")
    wid = ss * 2 + cc
    base0 = wid * EPT
    lanes = lax.iota(jnp.int32, 16)

    neg = jnp.full((16,), -3.0e38, jnp.float32)

    def initb(i, _):
        pm_l[pl.ds(i * 16, 16)] = neg
        return 0

    lax.fori_loop(0, NP // 16, initb, 0)

    bufs = ((sd0, qr0, kr0, scb0, gq0, gk0, st0, sdm0),
            (sd1, qr1, kr1, scb1, gq1, gk1, st1, sdm1))

    pltpu.sync_copy(sd_hbm.at[wid * NCH], sd0)
    pltpu.async_copy(sd_hbm.at[wid * NCH + 1], sd1, sdm1)
    pltpu.async_copy(q_hbm.at[sd0.at[1]], qr0, gq0)
    pltpu.async_copy(k_hbm.at[sd0.at[0]], kr0, gk0)

    def outer(g2, _):
        for b in range(2):
            sdb, qrows, krows, scb, gq, gk, st, sdm = bufs[b]
            nsdb, nqrows, nkrows, _, ngq, ngk, _, nsdm = bufs[1 - b]
            ci = g2 * 2 + b
            base = base0 + ci * C

            for t in range(C // 16):
                dstc[pl.ds(t * 16, 16)] = sdb[1, pl.ds(t * 16, 16)]

            pltpu.make_async_copy(q_hbm.at[sdb.at[1]], qrows, gq).wait()
            pltpu.make_async_copy(k_hbm.at[sdb.at[0]], krows, gk).wait()

            @pl.when(ci + 2 < NCH)
            def _():
                pltpu.async_copy(sd_hbm.at[wid * NCH + ci + 2], sdb, sdm)

            @pl.when(ci + 1 < NCH)
            def _():
                pltpu.make_async_copy(
                    sd_hbm.at[wid * NCH + ci + 1], nsdb, nsdm).wait()
                pltpu.async_copy(q_hbm.at[nsdb.at[1]], nqrows, ngq)
                pltpu.async_copy(k_hbm.at[nsdb.at[0]], nkrows, ngk)

            @pl.when(ci >= 2)
            def _():
                pltpu.make_async_copy(
                    scb, score_hbm.at[pl.ds(base0, C)], st).wait()

            def grp(g, _):
                def dotj(j, scv):
                    i = g * 16 + j
                    a = qrows[i, pl.ds(0, 16)] * krows[i, pl.ds(0, 16)]
                    for dd in range(1, D // 16):
                        a = a + (qrows[i, pl.ds(dd * 16, 16)]
                                 * krows[i, pl.ds(dd * 16, 16)])
                    return jnp.where(lanes == j, jnp.sum(a), scv)

                scv = lax.fori_loop(0, 16, dotj, jnp.zeros((16,), jnp.float32),
                                    unroll=4)
                scb[pl.ds(g * 16, 16)] = scv
                dv = dstc[pl.ds(g * 16, 16)]

                def bodyw(_carry):
                    cur = plsc.load_gather(pm_l, [dv])
                    need = scv > cur
                    plsc.store_scatter(pm_l, [dv], scv, mask=need)
                    cur2 = plsc.load_gather(pm_l, [dv])
                    return jnp.any(scv > cur2)

                lax.while_loop(lambda carry: carry, bodyw, jnp.bool_(True))
                return 0

            lax.fori_loop(0, C // 16, grp, 0)
            pltpu.async_copy(scb, score_hbm.at[pl.ds(base, C)], st)
        return 0

    lax.fori_loop(0, NCH // 2, outer, 0)
    for b in range(2):
        sdb, qrows, krows, scb, gq, gk, st, sdm = bufs[b]
        pltpu.make_async_copy(scb, score_hbm.at[pl.ds(base0, C)], st).wait()
    pltpu.sync_copy(pm_l, pm_hbm.at[wid])


@functools.partial(
    pl.kernel,
    out_type=[
        jax.ShapeDtypeStruct((EP,), jnp.float32),
        jax.ShapeDtypeStruct((NTILES, NP), jnp.float32),
    ],
    mesh=_mesh(),
    compiler_params=_SC_PARAMS,
    scratch_types=[
        pltpu.VMEM((NP,), jnp.float32),
        pltpu.VMEM((NP,), jnp.float32),
        pltpu.VMEM((EPT,), jnp.float32),
        pltpu.VMEM((EPT,), jnp.int32),
        pltpu.VMEM((EPT,), jnp.float32),
    ],
)
def _s2(score_hbm, dst_hbm, m_hbm, e_hbm, ps_hbm, m_l, s_l, scb, dstb, eb):
    cc = lax.axis_index("c")
    ss = lax.axis_index("s")
    wid = ss * 2 + cc
    base0 = wid * EPT

    pltpu.sync_copy(m_hbm, m_l)

    zero = jnp.zeros((16,), jnp.float32)

    def zz(i, _):
        s_l[pl.ds(i * 16, 16)] = zero
        return 0

    lax.fori_loop(0, NP // 16, zz, 0)

    pltpu.sync_copy(score_hbm.at[pl.ds(base0, EPT)], scb)
    pltpu.sync_copy(dst_hbm.at[pl.ds(base0, EPT)], dstb)

    def grp(i, _):
        dv = dstb[pl.ds(i * 16, 16)]
        sv = scb[pl.ds(i * 16, 16)]
        mv = plsc.load_gather(m_l, [dv])
        ev = jnp.exp(sv - mv)
        eb[pl.ds(i * 16, 16)] = ev
        plsc.addupdate_scatter(s_l, [dv], ev)
        return 0

    lax.fori_loop(0, EPT // 16, grp, 0)

    pltpu.sync_copy(eb, e_hbm.at[pl.ds(base0, EPT)])
    pltpu.sync_copy(s_l, ps_hbm.at[wid])


@functools.partial(
    pl.kernel,
    out_type=jax.ShapeDtypeStruct((2, NP, D), jnp.float32),
    mesh=_mesh(),
    compiler_params=_SC_PARAMS,
    scratch_types=[
        pltpu.VMEM((2, C), jnp.int32),
        pltpu.VMEM((2, C), jnp.int32),
        pltpu.VMEM((C, D), jnp.float32),
        pltpu.VMEM((C, D), jnp.float32),
        pltpu.VMEM((C,), jnp.float32),
        pltpu.VMEM((C,), jnp.float32),
        pltpu.VMEM((C,), jnp.int32),
        pltpu.VMEM((C,), jnp.int32),
        pltpu.VMEM((NP,), jnp.float32),
        pltpu.VMEM_SHARED((NP, D), jnp.float32),
        pltpu.SemaphoreType.DMA,
        pltpu.SemaphoreType.DMA,
        pltpu.SemaphoreType.DMA,
        pltpu.SemaphoreType.DMA,
        pltpu.SemaphoreType.DMA,
        pltpu.SemaphoreType.DMA,
        pltpu.SemaphoreType.DMA,
        pltpu.SemaphoreType.DMA,
    ],
)
def _s3(e_hbm, s_hbm, sd_hbm, v_hbm, aggp_hbm,
        sd0, sd1, vr0, vr1, ab0, ab1, dc0, dc1, s_l, agg,
        gv0, gv1, sc0, sc1, sdm0, sdm1, em0, em1):
    cc = lax.axis_index("c")
    ss = lax.axis_index("s")
    wid = ss * 2 + cc
    base0 = wid * EPT
    rows_per_tile = NP // 16

    pltpu.sync_copy(s_hbm, s_l)

    zero = jnp.zeros((16,), jnp.float32)

    def zr(i, _):
        for dd in range(D // 16):
            vr0[i, pl.ds(dd * 16, 16)] = zero
        return 0

    lax.fori_loop(0, C, zr, 0)
    for jj in range(rows_per_tile // C):
        pltpu.sync_copy(vr0, agg.at[pl.ds(ss * rows_per_tile + jj * C, C)])
    plsc.subcore_barrier()

    bufs = ((sd0, vr0, ab0, dc0, gv0, sc0, sdm0, em0),
            (sd1, vr1, ab1, dc1, gv1, sc1, sdm1, em1))

    pltpu.sync_copy(sd_hbm.at[wid * NCH], sd0)
    pltpu.async_copy(sd_hbm.at[wid * NCH + 1], sd1, sdm1)
    pltpu.sync_copy(e_hbm.at[pl.ds(base0, C)], ab0)
    pltpu.async_copy(e_hbm.at[pl.ds(base0 + C, C)], ab1, em1)
    pltpu.async_copy(v_hbm.at[sd0.at[0]], vr0, gv0)

    def outer(g2, _):
        for b in range(2):
            sdb, vrows, ab, dc, gv, sc, sdm, em = bufs[b]
            nsdb, nvrows, nab, ndc, ngv, nsc, nsdm, nem = bufs[1 - b]
            ci = g2 * 2 + b
            base = base0 + ci * C

            for t in range(C // 16):
                dc[pl.ds(t * 16, 16)] = sdb[1, pl.ds(t * 16, 16)]

            pltpu.make_async_copy(v_hbm.at[sdb.at[0]], vrows, gv).wait()

            @pl.when(ci + 2 < NCH)
            def _():
                pltpu.async_copy(sd_hbm.at[wid * NCH + ci + 2], sdb, sdm)

            @pl.when((ci + 1 < NCH) & (ci >= 1))
            def _():
                pltpu.make_async_copy(nvrows, agg.at[ndc], nsc).wait()

            @pl.when(ci + 1 < NCH)
            def _():
                pltpu.make_async_copy(
                    sd_hbm.at[wid * NCH + ci + 1], nsdb, nsdm).wait()
                pltpu.async_copy(v_hbm.at[nsdb.at[0]], nvrows, ngv)

            @pl.when(ci >= 1)
            def _():
                pltpu.make_async_copy(
                    e_hbm.at[pl.ds(base0, C)], ab, em).wait()

            def grp(g, _):
                dv = dc[pl.ds(g * 16, 16)]
                sv = plsc.load_gather(s_l, [dv])
                av = ab[pl.ds(g * 16, 16)] / sv
                for j in range(16):
                    i = g * 16 + j
                    a = av[j]
                    for dd in range(D // 16):
                        vrows[i, pl.ds(dd * 16, 16)] = (
                            vrows[i, pl.ds(dd * 16, 16)] * a)
                return 0

            lax.fori_loop(0, C // 16, grp, 0)
            pltpu.async_copy(vrows, agg.at[dc], sc, add=True)

            @pl.when(ci + 2 < NCH)
            def _():
                pltpu.async_copy(
                    e_hbm.at[pl.ds(base0 + (ci + 2) * C, C)], ab, em)
        return 0

    lax.fori_loop(0, NCH // 2, outer, 0)
    for b in range(2):
        sdb, vrows, ab, dc, gv, sc, sdm, em = bufs[b]
        pltpu.make_async_copy(vrows, agg.at[dc], sc).wait()
    plsc.subcore_barrier()
    pltpu.sync_copy(
        agg.at[pl.ds(ss * rows_per_tile, rows_per_tile)],
        aggp_hbm.at[cc, pl.ds(ss * rows_per_tile, rows_per_tile)],
    )


def kernel(x, edge_index, W_proj, b_proj, g1, b1, Wq1, Wk1, Wv1, tau1,
           Wq2, Wk2, Wv2, tau2, Wq3, Wk3, Wv3, tau3, W_ctx, b_ctx, g2, b2):
    f32 = jnp.float32
    i32 = jnp.int32

    src0 = edge_index[0].astype(i32)
    dst0 = edge_index[1].astype(i32)
    loop = jnp.arange(N, dtype=i32)
    n_epad = EP - E - N
    padi = N + (jnp.arange(n_epad, dtype=i32) % (NP - N))
    src = jnp.concatenate([src0, loop, padi])
    dst = jnp.concatenate([dst0, loop, padi])
    sd = jnp.stack(
        [src.reshape(NTILES * NCH, C), dst.reshape(NTILES * NCH, C)], axis=1)

    xp = jnp.pad(x.astype(f32), ((0, NP - N), (0, 0)))

    def row(a):
        return a.astype(f32).reshape(1, -1)

    h0 = _proj_ln(xp, W_proj.astype(f32), row(b_proj), row(g1), row(b1))

    layers = [
        (Wq1, Wk1, Wv1, tau1),
        (Wq2, Wk2, Wv2, tau2),
        (Wq3, Wk3, Wv3, tau3),
    ]

    hs = [h0]
    aggp = None
    for (wq, wk, wv, tau) in layers:
        tau2d = tau.astype(f32).reshape(1, 1)
        if aggp is None:
            q, k, v = _qkv1(tau2d, h0, wq.astype(f32), wk.astype(f32),
                            wv.astype(f32))
        else:
            hprev, q, k, v = _qkv2(tau2d, aggp[0], aggp[1], wq.astype(f32),
                                   wk.astype(f32), wv.astype(f32))
            hs.append(hprev)
        score, pm = _s1(q, k, sd)
        m = _colreduce(pm, _colmax_body).reshape(NP)
        e, ps = _s2(score, dst, m)
        s = _colreduce(ps, _colsum_body).reshape(NP)
        aggp = _s3(e, s, sd, v)

    out = _final(hs[0], hs[1], hs[2], aggp[0], aggp[1], W_ctx.astype(f32),
                 row(b_ctx), row(g2), row(b2))
    return out[:N]


# speedup vs baseline: 20.0098x; 1.0070x over previous
import functools

import jax
import jax.numpy as jnp
from jax import lax
from jax.experimental import pallas as pl
from jax.experimental.pallas import tpu as pltpu
from jax.experimental.pallas import tpu_sc as plsc

N = 10000
D = 128
H = 128
OUT = 768
E = 320000

NP = 10240
NTILES = 32
C = 128
NCH = 82
EPT = NCH * C
EP = NTILES * EPT
LN_EPS = 1e-5

_SC_PARAMS = pltpu.CompilerParams(needs_layout_passes=False)


def _mesh():
    return plsc.VectorSubcoreMesh(
        core_axis_name="c", subcore_axis_name="s", num_cores=2, num_subcores=16
    )


def _gelu(x):
    return 0.5 * x * (1.0 + lax.erf(x * (2.0 ** -0.5)))


def _proj_ln_body(x_ref, w_ref, b_ref, g_ref, bb_ref, o_ref):
    y = jnp.dot(x_ref[...], w_ref[...], preferred_element_type=jnp.float32)
    y = y + b_ref[...]
    mu = jnp.mean(y, axis=-1, keepdims=True)
    var = jnp.mean(jnp.square(y - mu), axis=-1, keepdims=True)
    o_ref[...] = (y - mu) * lax.rsqrt(var + LN_EPS) * g_ref[...] + bb_ref[...]


def _proj_ln(x, w, b, g, bb):
    BN = 1024
    return pl.pallas_call(
        _proj_ln_body,
        grid=(NP // BN,),
        in_specs=[
            pl.BlockSpec((BN, D), lambda i: (i, 0)),
            pl.BlockSpec((D, H), lambda i: (0, 0)),
            pl.BlockSpec((1, H), lambda i: (0, 0)),
            pl.BlockSpec((1, H), lambda i: (0, 0)),
            pl.BlockSpec((1, H), lambda i: (0, 0)),
        ],
        out_specs=pl.BlockSpec((BN, H), lambda i: (i, 0)),
        out_shape=jax.ShapeDtypeStruct((NP, H), jnp.float32),
    )(x, w, b, g, bb)


def _qkv1_body(tau_ref, h_ref, wq_ref, wk_ref, wv_ref, q_ref, k_ref, v_ref):
    h = h_ref[...]
    scale = 1.0 / jnp.maximum(tau_ref[0, 0], 0.001)
    q_ref[...] = jnp.dot(h, wq_ref[...], preferred_element_type=jnp.float32) * scale
    k_ref[...] = jnp.dot(h, wk_ref[...], preferred_element_type=jnp.float32)
    v_ref[...] = jnp.dot(h, wv_ref[...], preferred_element_type=jnp.float32)


def _qkv1(tau, h, wq, wk, wv):
    BN = 1024
    return pl.pallas_call(
        _qkv1_body,
        grid=(NP // BN,),
        in_specs=[
            pl.BlockSpec(memory_space=pltpu.SMEM),
            pl.BlockSpec((BN, H), lambda i: (i, 0)),
            pl.BlockSpec((H, H), lambda i: (0, 0)),
            pl.BlockSpec((H, H), lambda i: (0, 0)),
            pl.BlockSpec((H, H), lambda i: (0, 0)),
        ],
        out_specs=[
            pl.BlockSpec((BN, H), lambda i: (i, 0)),
            pl.BlockSpec((BN, H), lambda i: (i, 0)),
            pl.BlockSpec((BN, H), lambda i: (i, 0)),
        ],
        out_shape=[
            jax.ShapeDtypeStruct((NP, H), jnp.float32),
            jax.ShapeDtypeStruct((NP, H), jnp.float32),
            jax.ShapeDtypeStruct((NP, H), jnp.float32),
        ],
    )(tau, h, wq, wk, wv)


def _qkv2_body(tau_ref, a0_ref, a1_ref, wq_ref, wk_ref, wv_ref,
               h_ref, q_ref, k_ref, v_ref):
    h = _gelu(a0_ref[...] + a1_ref[...])
    h_ref[...] = h
    scale = 1.0 / jnp.maximum(tau_ref[0, 0], 0.001)
    q_ref[...] = jnp.dot(h, wq_ref[...], preferred_element_type=jnp.float32) * scale
    k_ref[...] = jnp.dot(h, wk_ref[...], preferred_element_type=jnp.float32)
    v_ref[...] = jnp.dot(h, wv_ref[...], preferred_element_type=jnp.float32)


def _qkv2(tau, a0, a1, wq, wk, wv):
    BN = 1024
    return pl.pallas_call(
        _qkv2_body,
        grid=(NP // BN,),
        in_specs=[
            pl.BlockSpec(memory_space=pltpu.SMEM),
            pl.BlockSpec((BN, H), lambda i: (i, 0)),
            pl.BlockSpec((BN, H), lambda i: (i, 0)),
            pl.BlockSpec((H, H), lambda i: (0, 0)),
            pl.BlockSpec((H, H), lambda i: (0, 0)),
            pl.BlockSpec((H, H), lambda i: (0, 0)),
        ],
        out_specs=[
            pl.BlockSpec((BN, H), lambda i: (i, 0)),
            pl.BlockSpec((BN, H), lambda i: (i, 0)),
            pl.BlockSpec((BN, H), lambda i: (i, 0)),
            pl.BlockSpec((BN, H), lambda i: (i, 0)),
        ],
        out_shape=[
            jax.ShapeDtypeStruct((NP, H), jnp.float32),
            jax.ShapeDtypeStruct((NP, H), jnp.float32),
            jax.ShapeDtypeStruct((NP, H), jnp.float32),
            jax.ShapeDtypeStruct((NP, H), jnp.float32),
        ],
    )(tau, a0, a1, wq, wk, wv)


def _colmax_body(p_ref, o_ref):
    o_ref[...] = jnp.max(p_ref[...], axis=0, keepdims=True)


def _colsum_body(p_ref, o_ref):
    o_ref[...] = jnp.sum(p_ref[...], axis=0, keepdims=True)


def _colreduce(p, body):
    BC = 1280
    return pl.pallas_call(
        body,
        grid=(NP // BC,),
        in_specs=[pl.BlockSpec((NTILES, BC), lambda i: (0, i))],
        out_specs=pl.BlockSpec((1, BC), lambda i: (0, i)),
        out_shape=jax.ShapeDtypeStruct((1, NP), jnp.float32),
    )(p)


def _final_body(h0_ref, h1_ref, h2_ref, a0_ref, a1_ref, w_ref, b_ref,
                g_ref, bb_ref, o_ref):
    h3 = _gelu(a0_ref[...] + a1_ref[...])
    w = w_ref[...]
    acc = jnp.dot(h0_ref[...], w[0:H], preferred_element_type=jnp.float32)
    acc = acc + jnp.dot(h1_ref[...], w[H:2 * H], preferred_element_type=jnp.float32)
    acc = acc + jnp.dot(h2_ref[...], w[2 * H:3 * H], preferred_element_type=jnp.float32)
    acc = acc + jnp.dot(h3, w[3 * H:4 * H], preferred_element_type=jnp.float32)
    y = _gelu(acc + b_ref[...])
    mu = jnp.mean(y, axis=-1, keepdims=True)
    var = jnp.mean(jnp.square(y - mu), axis=-1, keepdims=True)
    o_ref[...] = (y - mu) * lax.rsqrt(var + LN_EPS) * g_ref[...] + bb_ref[...]


def _final(h0, h1, h2, a0, a1, w, b, g, bb):
    BN = 512
    return pl.pallas_call(
        _final_body,
        grid=(NP // BN,),
        in_specs=[
            pl.BlockSpec((BN, H), lambda i: (i, 0)),
            pl.BlockSpec((BN, H), lambda i: (i, 0)),
            pl.BlockSpec((BN, H), lambda i: (i, 0)),
            pl.BlockSpec((BN, H), lambda i: (i, 0)),
            pl.BlockSpec((BN, H), lambda i: (i, 0)),
            pl.BlockSpec((4 * H, OUT), lambda i: (0, 0)),
            pl.BlockSpec((1, OUT), lambda i: (0, 0)),
            pl.BlockSpec((1, OUT), lambda i: (0, 0)),
            pl.BlockSpec((1, OUT), lambda i: (0, 0)),
        ],
        out_specs=pl.BlockSpec((BN, OUT), lambda i: (i, 0)),
        out_shape=jax.ShapeDtypeStruct((NP, OUT), jnp.float32),
    )(h0, h1, h2, a0, a1, w, b, g, bb)


@functools.partial(
    pl.kernel,
    out_type=[
        jax.ShapeDtypeStruct((EP,), jnp.float32),
        jax.ShapeDtypeStruct((NTILES, NP), jnp.float32),
    ],
    mesh=_mesh(),
    compiler_params=_SC_PARAMS,
    scratch_types=[
        pltpu.VMEM((2, C), jnp.int32),
        pltpu.VMEM((2, C), jnp.int32),
        pltpu.VMEM((C, D), jnp.float32),
        pltpu.VMEM((C, D), jnp.float32),
        pltpu.VMEM((C, D), jnp.float32),
        pltpu.VMEM((C, D), jnp.float32),
        pltpu.VMEM((C,), jnp.float32),
        pltpu.VMEM((C,), jnp.float32),
        pltpu.VMEM((C,), jnp.int32),
        pltpu.VMEM((NP,), jnp.float32),
        pltpu.SemaphoreType.DMA,
        pltpu.SemaphoreType.DMA,
        pltpu.SemaphoreType.DMA,
        pltpu.SemaphoreType.DMA,
        pltpu.SemaphoreType.DMA,
        pltpu.SemaphoreType.DMA,
        pltpu.SemaphoreType.DMA,
        pltpu.SemaphoreType.DMA,
    ],
)
def _s1(q_hbm, k_hbm, sd_hbm, score_hbm, pm_hbm,
        sd0, sd1, qr0, kr0, qr1, kr1, scb0, scb1, dstc, pm_l,
        gq0, gk0, gq1, gk1, st0, st1, sdm0, sdm1):
    cc = lax.axis_index("c")
    ss = lax.axis_index("s")
    wid = ss * 2 + cc
    base0 = wid * EPT
    lanes = lax.iota(jnp.int32, 16)

    neg = jnp.full((16,), -3.0e38, jnp.float32)

    def initb(i, _):
        pm_l[pl.ds(i * 16, 16)] = neg
        return 0

    lax.fori_loop(0, NP // 16, initb, 0)

    bufs = ((sd0, qr0, kr0, scb0, gq0, gk0, st0, sdm0),
            (sd1, qr1, kr1, scb1, gq1, gk1, st1, sdm1))

    pltpu.sync_copy(sd_hbm.at[wid * NCH], sd0)
    pltpu.async_copy(sd_hbm.at[wid * NCH + 1], sd1, sdm1)
    pltpu.async_copy(q_hbm.at[sd0.at[1]], qr0, gq0)
    pltpu.async_copy(k_hbm.at[sd0.at[0]], kr0, gk0)

    def outer(g2, _):
        for b in range(2):
            sdb, qrows, krows, scb, gq, gk, st, sdm = bufs[b]
            nsdb, nqrows, nkrows, _, ngq, ngk, _, nsdm = bufs[1 - b]
            ci = g2 * 2 + b
            base = base0 + ci * C

            for t in range(C // 16):
                dstc[pl.ds(t * 16, 16)] = sdb[1, pl.ds(t * 16, 16)]

            pltpu.make_async_copy(q_hbm.at[sdb.at[1]], qrows, gq).wait()
            pltpu.make_async_copy(k_hbm.at[sdb.at[0]], krows, gk).wait()

            @pl.when(ci + 2 < NCH)
            def _():
                pltpu.async_copy(sd_hbm.at[wid * NCH + ci + 2], sdb, sdm)

            @pl.when(ci + 1 < NCH)
            def _():
                pltpu.make_async_copy(
                    sd_hbm.at[wid * NCH + ci + 1], nsdb, nsdm).wait()
                pltpu.async_copy(q_hbm.at[nsdb.at[1]], nqrows, ngq)
                pltpu.async_copy(k_hbm.at[nsdb.at[0]], nkrows, ngk)

            @pl.when(ci >= 2)
            def _():
                pltpu.make_async_copy(
                    scb, score_hbm.at[pl.ds(base0, C)], st).wait()

            def grp(g, _):
                def dotj(j, scv):
                    i = g * 16 + j
                    a = qrows[i, pl.ds(0, 16)] * krows[i, pl.ds(0, 16)]
                    for dd in range(1, D // 16):
                        a = a + (qrows[i, pl.ds(dd * 16, 16)]
                                 * krows[i, pl.ds(dd * 16, 16)])
                    return jnp.where(lanes == j, jnp.sum(a), scv)

                scv = lax.fori_loop(0, 16, dotj, jnp.zeros((16,), jnp.float32),
                                    unroll=4)
                scb[pl.ds(g * 16, 16)] = scv
                dv = dstc[pl.ds(g * 16, 16)]

                def bodyw(_carry):
                    cur = plsc.load_gather(pm_l, [dv])
                    need = scv > cur
                    plsc.store_scatter(pm_l, [dv], scv, mask=need)
                    cur2 = plsc.load_gather(pm_l, [dv])
                    return jnp.any(scv > cur2)

                lax.while_loop(lambda carry: carry, bodyw, jnp.bool_(True))
                return 0

            lax.fori_loop(0, C // 16, grp, 0)
            pltpu.async_copy(scb, score_hbm.at[pl.ds(base, C)], st)
        return 0

    lax.fori_loop(0, NCH // 2, outer, 0)
    for b in range(2):
        sdb, qrows, krows, scb, gq, gk, st, sdm = bufs[b]
        pltpu.make_async_copy(scb, score_hbm.at[pl.ds(base0, C)], st).wait()
    pltpu.sync_copy(pm_l, pm_hbm.at[wid])


@functools.partial(
    pl.kernel,
    out_type=[
        jax.ShapeDtypeStruct((EP,), jnp.float32),
        jax.ShapeDtypeStruct((NTILES, NP), jnp.float32),
    ],
    mesh=_mesh(),
    compiler_params=_SC_PARAMS,
    scratch_types=[
        pltpu.VMEM((NP,), jnp.float32),
        pltpu.VMEM((NP,), jnp.float32),
        pltpu.VMEM((EPT,), jnp.float32),
        pltpu.VMEM((EPT,), jnp.int32),
        pltpu.VMEM((EPT,), jnp.float32),
        pltpu.VMEM((NTILES, NP // 16), jnp.float32),
        pltpu.VMEM((NP // 16,), jnp.float32),
        pltpu.VMEM_SHARED((NP,), jnp.float32),
    ],
)
def _s2(score_hbm, dst_hbm, pm_hbm, e_hbm, ps_hbm,
        m_l, s_l, scb, dstb, eb, pmb, mslice, m_sh):
    cc = lax.axis_index("c")
    ss = lax.axis_index("s")
    wid = ss * 2 + cc
    base0 = wid * EPT
    npt = NP // 16

    pltpu.sync_copy(pm_hbm.at[pl.ds(0, NTILES), pl.ds(ss * npt, npt)], pmb)

    def redg(g, _):
        acc = pmb[0, pl.ds(g * 16, 16)]
        for r in range(1, NTILES):
            acc = jnp.maximum(acc, pmb[r, pl.ds(g * 16, 16)])
        mslice[pl.ds(g * 16, 16)] = acc
        return 0

    lax.fori_loop(0, npt // 16, redg, 0)
    pltpu.sync_copy(mslice, m_sh.at[pl.ds(ss * npt, npt)])
    plsc.subcore_barrier()
    pltpu.sync_copy(m_sh, m_l)

    zero = jnp.zeros((16,), jnp.float32)

    def zz(i, _):
        s_l[pl.ds(i * 16, 16)] = zero
        return 0

    lax.fori_loop(0, NP // 16, zz, 0)

    pltpu.sync_copy(score_hbm.at[pl.ds(base0, EPT)], scb)
    pltpu.sync_copy(dst_hbm.at[pl.ds(base0, EPT)], dstb)

    def grp(i, _):
        dv = dstb[pl.ds(i * 16, 16)]
        sv = scb[pl.ds(i * 16, 16)]
        mv = plsc.load_gather(m_l, [dv])
        ev = jnp.exp(sv - mv)
        eb[pl.ds(i * 16, 16)] = ev
        plsc.addupdate_scatter(s_l, [dv], ev)
        return 0

    lax.fori_loop(0, EPT // 16, grp, 0)

    pltpu.sync_copy(eb, e_hbm.at[pl.ds(base0, EPT)])
    pltpu.sync_copy(s_l, ps_hbm.at[wid])


@functools.partial(
    pl.kernel,
    out_type=jax.ShapeDtypeStruct((2, NP, D), jnp.float32),
    mesh=_mesh(),
    compiler_params=_SC_PARAMS,
    scratch_types=[
        pltpu.VMEM((2, C), jnp.int32),
        pltpu.VMEM((2, C), jnp.int32),
        pltpu.VMEM((C, D), jnp.float32),
        pltpu.VMEM((C, D), jnp.float32),
        pltpu.VMEM((C,), jnp.float32),
        pltpu.VMEM((C,), jnp.float32),
        pltpu.VMEM((C,), jnp.int32),
        pltpu.VMEM((C,), jnp.int32),
        pltpu.VMEM((NP,), jnp.float32),
        pltpu.VMEM_SHARED((NP, D), jnp.float32),
        pltpu.SemaphoreType.DMA,
        pltpu.SemaphoreType.DMA,
        pltpu.SemaphoreType.DMA,
        pltpu.SemaphoreType.DMA,
        pltpu.SemaphoreType.DMA,
        pltpu.SemaphoreType.DMA,
        pltpu.SemaphoreType.DMA,
        pltpu.SemaphoreType.DMA,
    ],
)
def _s3(e_hbm, s_hbm, sd_hbm, v_hbm, aggp_hbm,
        sd0, sd1, vr0, vr1, ab0, ab1, dc0, dc1, s_l, agg,
        gv0, gv1, sc0, sc1, sdm0, sdm1, em0, em1):
    cc = lax.axis_index("c")
    ss = lax.axis_index("s")
    wid = ss * 2 + cc
    base0 = wid * EPT
    rows_per_tile = NP // 16

    pltpu.sync_copy(s_hbm, s_l)

    zero = jnp.zeros((16,), jnp.float32)

    def zr(i, _):
        for dd in range(D // 16):
            vr0[i, pl.ds(dd * 16, 16)] = zero
        return 0

    lax.fori_loop(0, C, zr, 0)
    for jj in range(rows_per_tile // C):
        pltpu.sync_copy(vr0, agg.at[pl.ds(ss * rows_per_tile + jj * C, C)])
    plsc.subcore_barrier()

    bufs = ((sd0, vr0, ab0, dc0, gv0, sc0, sdm0, em0),
            (sd1, vr1, ab1, dc1, gv1, sc1, sdm1, em1))

    pltpu.sync_copy(sd_hbm.at[wid * NCH], sd0)
    pltpu.async_copy(sd_hbm.at[wid * NCH + 1], sd1, sdm1)
    pltpu.sync_copy(e_hbm.at[pl.ds(base0, C)], ab0)
    pltpu.async_copy(e_hbm.at[pl.ds(base0 + C, C)], ab1, em1)
    pltpu.async_copy(v_hbm.at[sd0.at[0]], vr0, gv0)

    def outer(g2, _):
        for b in range(2):
            sdb, vrows, ab, dc, gv, sc, sdm, em = bufs[b]
            nsdb, nvrows, nab, ndc, ngv, nsc, nsdm, nem = bufs[1 - b]
            ci = g2 * 2 + b
            base = base0 + ci * C

            for t in range(C // 16):
                dc[pl.ds(t * 16, 16)] = sdb[1, pl.ds(t * 16, 16)]

            pltpu.make_async_copy(v_hbm.at[sdb.at[0]], vrows, gv).wait()

            @pl.when(ci + 2 < NCH)
            def _():
                pltpu.async_copy(sd_hbm.at[wid * NCH + ci + 2], sdb, sdm)

            @pl.when((ci + 1 < NCH) & (ci >= 1))
            def _():
                pltpu.make_async_copy(nvrows, agg.at[ndc], nsc).wait()

            @pl.when(ci + 1 < NCH)
            def _():
                pltpu.make_async_copy(
                    sd_hbm.at[wid * NCH + ci + 1], nsdb, nsdm).wait()
                pltpu.async_copy(v_hbm.at[nsdb.at[0]], nvrows, ngv)

            @pl.when(ci >= 1)
            def _():
                pltpu.make_async_copy(
                    e_hbm.at[pl.ds(base0, C)], ab, em).wait()

            def grp(g, _):
                dv = dc[pl.ds(g * 16, 16)]
                sv = plsc.load_gather(s_l, [dv])
                av = ab[pl.ds(g * 16, 16)] / sv
                for j in range(16):
                    i = g * 16 + j
                    a = av[j]
                    for dd in range(D // 16):
                        vrows[i, pl.ds(dd * 16, 16)] = (
                            vrows[i, pl.ds(dd * 16, 16)] * a)
                return 0

            lax.fori_loop(0, C // 16, grp, 0)
            pltpu.async_copy(vrows, agg.at[dc], sc, add=True)

            @pl.when(ci + 2 < NCH)
            def _():
                pltpu.async_copy(
                    e_hbm.at[pl.ds(base0 + (ci + 2) * C, C)], ab, em)
        return 0

    lax.fori_loop(0, NCH // 2, outer, 0)
    for b in range(2):
        sdb, vrows, ab, dc, gv, sc, sdm, em = bufs[b]
        pltpu.make_async_copy(vrows, agg.at[dc], sc).wait()
    plsc.subcore_barrier()
    pltpu.sync_copy(
        agg.at[pl.ds(ss * rows_per_tile, rows_per_tile)],
        aggp_hbm.at[cc, pl.ds(ss * rows_per_tile, rows_per_tile)],
    )


def kernel(x, edge_index, W_proj, b_proj, g1, b1, Wq1, Wk1, Wv1, tau1,
           Wq2, Wk2, Wv2, tau2, Wq3, Wk3, Wv3, tau3, W_ctx, b_ctx, g2, b2):
    f32 = jnp.float32
    i32 = jnp.int32

    src0 = edge_index[0].astype(i32)
    dst0 = edge_index[1].astype(i32)
    loop = jnp.arange(N, dtype=i32)
    n_epad = EP - E - N
    padi = N + (jnp.arange(n_epad, dtype=i32) % (NP - N))
    src = jnp.concatenate([src0, loop, padi])
    dst = jnp.concatenate([dst0, loop, padi])
    sd = jnp.stack(
        [src.reshape(NTILES * NCH, C), dst.reshape(NTILES * NCH, C)], axis=1)

    xp = jnp.pad(x.astype(f32), ((0, NP - N), (0, 0)))

    def row(a):
        return a.astype(f32).reshape(1, -1)

    h0 = _proj_ln(xp, W_proj.astype(f32), row(b_proj), row(g1), row(b1))

    layers = [
        (Wq1, Wk1, Wv1, tau1),
        (Wq2, Wk2, Wv2, tau2),
        (Wq3, Wk3, Wv3, tau3),
    ]

    hs = [h0]
    aggp = None
    for (wq, wk, wv, tau) in layers:
        tau2d = tau.astype(f32).reshape(1, 1)
        if aggp is None:
            q, k, v = _qkv1(tau2d, h0, wq.astype(f32), wk.astype(f32),
                            wv.astype(f32))
        else:
            hprev, q, k, v = _qkv2(tau2d, aggp[0], aggp[1], wq.astype(f32),
                                   wk.astype(f32), wv.astype(f32))
            hs.append(hprev)
        score, pm = _s1(q, k, sd)
        e, ps = _s2(score, dst, pm)
        s = _colreduce(ps, _colsum_body).reshape(NP)
        aggp = _s3(e, s, sd, v)

    out = _final(hs[0], hs[1], hs[2], aggp[0], aggp[1], W_ctx.astype(f32),
                 row(b_ctx), row(g2), row(b2))
    return out[:N]


# speedup vs baseline: 20.2611x; 1.0126x over previous
import functools

import jax
import jax.numpy as jnp
from jax import lax
from jax.experimental import pallas as pl
from jax.experimental.pallas import tpu as pltpu
from jax.experimental.pallas import tpu_sc as plsc

N = 10000
D = 128
H = 128
OUT = 768
E = 320000

NP = 10240
NTILES = 32
C = 128
NCH = 82
EPT = NCH * C
EP = NTILES * EPT
LN_EPS = 1e-5

_SC_PARAMS = pltpu.CompilerParams(needs_layout_passes=False)


def _mesh():
    return plsc.VectorSubcoreMesh(
        core_axis_name="c", subcore_axis_name="s", num_cores=2, num_subcores=16
    )


def _gelu(x):
    return 0.5 * x * (1.0 + lax.erf(x * (2.0 ** -0.5)))


def _proj_ln_body(x_ref, w_ref, b_ref, g_ref, bb_ref, o_ref):
    y = jnp.dot(x_ref[...], w_ref[...], preferred_element_type=jnp.float32)
    y = y + b_ref[...]
    mu = jnp.mean(y, axis=-1, keepdims=True)
    var = jnp.mean(jnp.square(y - mu), axis=-1, keepdims=True)
    o_ref[...] = (y - mu) * lax.rsqrt(var + LN_EPS) * g_ref[...] + bb_ref[...]


def _proj_ln(x, w, b, g, bb):
    BN = 1024
    return pl.pallas_call(
        _proj_ln_body,
        grid=(NP // BN,),
        in_specs=[
            pl.BlockSpec((BN, D), lambda i: (i, 0)),
            pl.BlockSpec((D, H), lambda i: (0, 0)),
            pl.BlockSpec((1, H), lambda i: (0, 0)),
            pl.BlockSpec((1, H), lambda i: (0, 0)),
            pl.BlockSpec((1, H), lambda i: (0, 0)),
        ],
        out_specs=pl.BlockSpec((BN, H), lambda i: (i, 0)),
        out_shape=jax.ShapeDtypeStruct((NP, H), jnp.float32),
    )(x, w, b, g, bb)


def _qkv1_body(tau_ref, h_ref, wq_ref, wk_ref, wv_ref, q_ref, k_ref, v_ref):
    h = h_ref[...]
    scale = 1.0 / jnp.maximum(tau_ref[0, 0], 0.001)
    q_ref[...] = jnp.dot(h, wq_ref[...], preferred_element_type=jnp.float32) * scale
    k_ref[...] = jnp.dot(h, wk_ref[...], preferred_element_type=jnp.float32)
    v_ref[...] = jnp.dot(h, wv_ref[...], preferred_element_type=jnp.float32)


def _qkv1(tau, h, wq, wk, wv):
    BN = 1024
    return pl.pallas_call(
        _qkv1_body,
        grid=(NP // BN,),
        in_specs=[
            pl.BlockSpec(memory_space=pltpu.SMEM),
            pl.BlockSpec((BN, H), lambda i: (i, 0)),
            pl.BlockSpec((H, H), lambda i: (0, 0)),
            pl.BlockSpec((H, H), lambda i: (0, 0)),
            pl.BlockSpec((H, H), lambda i: (0, 0)),
        ],
        out_specs=[
            pl.BlockSpec((BN, H), lambda i: (i, 0)),
            pl.BlockSpec((BN, H), lambda i: (i, 0)),
            pl.BlockSpec((BN, H), lambda i: (i, 0)),
        ],
        out_shape=[
            jax.ShapeDtypeStruct((NP, H), jnp.float32),
            jax.ShapeDtypeStruct((NP, H), jnp.float32),
            jax.ShapeDtypeStruct((NP, H), jnp.float32),
        ],
    )(tau, h, wq, wk, wv)


def _qkv2_body(tau_ref, a0_ref, a1_ref, wq_ref, wk_ref, wv_ref,
               h_ref, q_ref, k_ref, v_ref):
    h = _gelu(a0_ref[...] + a1_ref[...])
    h_ref[...] = h
    scale = 1.0 / jnp.maximum(tau_ref[0, 0], 0.001)
    q_ref[...] = jnp.dot(h, wq_ref[...], preferred_element_type=jnp.float32) * scale
    k_ref[...] = jnp.dot(h, wk_ref[...], preferred_element_type=jnp.float32)
    v_ref[...] = jnp.dot(h, wv_ref[...], preferred_element_type=jnp.float32)


def _qkv2(tau, a0, a1, wq, wk, wv):
    BN = 1024
    return pl.pallas_call(
        _qkv2_body,
        grid=(NP // BN,),
        in_specs=[
            pl.BlockSpec(memory_space=pltpu.SMEM),
            pl.BlockSpec((BN, H), lambda i: (i, 0)),
            pl.BlockSpec((BN, H), lambda i: (i, 0)),
            pl.BlockSpec((H, H), lambda i: (0, 0)),
            pl.BlockSpec((H, H), lambda i: (0, 0)),
            pl.BlockSpec((H, H), lambda i: (0, 0)),
        ],
        out_specs=[
            pl.BlockSpec((BN, H), lambda i: (i, 0)),
            pl.BlockSpec((BN, H), lambda i: (i, 0)),
            pl.BlockSpec((BN, H), lambda i: (i, 0)),
            pl.BlockSpec((BN, H), lambda i: (i, 0)),
        ],
        out_shape=[
            jax.ShapeDtypeStruct((NP, H), jnp.float32),
            jax.ShapeDtypeStruct((NP, H), jnp.float32),
            jax.ShapeDtypeStruct((NP, H), jnp.float32),
            jax.ShapeDtypeStruct((NP, H), jnp.float32),
        ],
    )(tau, a0, a1, wq, wk, wv)


def _colsum_body(p_ref, o_ref):
    o_ref[...] = jnp.sum(p_ref[...], axis=0, keepdims=True)


def _colreduce(p, body):
    BC = 1280
    return pl.pallas_call(
        body,
        grid=(NP // BC,),
        in_specs=[pl.BlockSpec((NTILES, BC), lambda i: (0, i))],
        out_specs=pl.BlockSpec((1, BC), lambda i: (0, i)),
        out_shape=jax.ShapeDtypeStruct((1, NP), jnp.float32),
    )(p)


def _final_body(h0_ref, h1_ref, h2_ref, a0_ref, a1_ref, w_ref, b_ref,
                g_ref, bb_ref, o_ref):
    h3 = _gelu(a0_ref[...] + a1_ref[...])
    w = w_ref[...]
    acc = jnp.dot(h0_ref[...], w[0:H], preferred_element_type=jnp.float32)
    acc = acc + jnp.dot(h1_ref[...], w[H:2 * H], preferred_element_type=jnp.float32)
    acc = acc + jnp.dot(h2_ref[...], w[2 * H:3 * H], preferred_element_type=jnp.float32)
    acc = acc + jnp.dot(h3, w[3 * H:4 * H], preferred_element_type=jnp.float32)
    y = _gelu(acc + b_ref[...])
    mu = jnp.mean(y, axis=-1, keepdims=True)
    var = jnp.mean(jnp.square(y - mu), axis=-1, keepdims=True)
    o_ref[...] = (y - mu) * lax.rsqrt(var + LN_EPS) * g_ref[...] + bb_ref[...]


def _final(h0, h1, h2, a0, a1, w, b, g, bb):
    BN = 512
    return pl.pallas_call(
        _final_body,
        grid=(NP // BN,),
        in_specs=[
            pl.BlockSpec((BN, H), lambda i: (i, 0)),
            pl.BlockSpec((BN, H), lambda i: (i, 0)),
            pl.BlockSpec((BN, H), lambda i: (i, 0)),
            pl.BlockSpec((BN, H), lambda i: (i, 0)),
            pl.BlockSpec((BN, H), lambda i: (i, 0)),
            pl.BlockSpec((4 * H, OUT), lambda i: (0, 0)),
            pl.BlockSpec((1, OUT), lambda i: (0, 0)),
            pl.BlockSpec((1, OUT), lambda i: (0, 0)),
            pl.BlockSpec((1, OUT), lambda i: (0, 0)),
        ],
        out_specs=pl.BlockSpec((BN, OUT), lambda i: (i, 0)),
        out_shape=jax.ShapeDtypeStruct((NP, OUT), jnp.float32),
    )(h0, h1, h2, a0, a1, w, b, g, bb)


@functools.partial(
    pl.kernel,
    out_type=[
        jax.ShapeDtypeStruct((EP,), jnp.float32),
        jax.ShapeDtypeStruct((NTILES, NP), jnp.float32),
    ],
    mesh=_mesh(),
    compiler_params=_SC_PARAMS,
    scratch_types=[
        pltpu.VMEM((2, C), jnp.int32),
        pltpu.VMEM((2, C), jnp.int32),
        pltpu.VMEM((C, D), jnp.float32),
        pltpu.VMEM((C, D), jnp.float32),
        pltpu.VMEM((C, D), jnp.float32),
        pltpu.VMEM((C, D), jnp.float32),
        pltpu.VMEM((C,), jnp.float32),
        pltpu.VMEM((C,), jnp.float32),
        pltpu.VMEM((C,), jnp.int32),
        pltpu.VMEM((NP,), jnp.float32),
        pltpu.SemaphoreType.DMA,
        pltpu.SemaphoreType.DMA,
        pltpu.SemaphoreType.DMA,
        pltpu.SemaphoreType.DMA,
        pltpu.SemaphoreType.DMA,
        pltpu.SemaphoreType.DMA,
        pltpu.SemaphoreType.DMA,
        pltpu.SemaphoreType.DMA,
    ],
)
def _s1(q_hbm, k_hbm, sd_hbm, score_hbm, pm_hbm,
        sd0, sd1, qr0, kr0, qr1, kr1, scb0, scb1, dstc, pm_l,
        gq0, gk0, gq1, gk1, st0, st1, sdm0, sdm1):
    cc = lax.axis_index("c")
    ss = lax.axis_index("s")
    wid = ss * 2 + cc
    base0 = wid * EPT
    lanes = lax.iota(jnp.int32, 16)

    neg = jnp.full((16,), -3.0e38, jnp.float32)

    def initb(i, _):
        pm_l[pl.ds(i * 16, 16)] = neg
        return 0

    lax.fori_loop(0, NP // 16, initb, 0)

    bufs = ((sd0, qr0, kr0, scb0, gq0, gk0, st0, sdm0),
            (sd1, qr1, kr1, scb1, gq1, gk1, st1, sdm1))

    pltpu.sync_copy(sd_hbm.at[wid * NCH], sd0)
    pltpu.async_copy(sd_hbm.at[wid * NCH + 1], sd1, sdm1)
    pltpu.async_copy(q_hbm.at[sd0.at[1]], qr0, gq0)
    pltpu.async_copy(k_hbm.at[sd0.at[0]], kr0, gk0)

    def outer(g2, _):
        for b in range(2):
            sdb, qrows, krows, scb, gq, gk, st, sdm = bufs[b]
            nsdb, nqrows, nkrows, _, ngq, ngk, _, nsdm = bufs[1 - b]
            ci = g2 * 2 + b
            base = base0 + ci * C

            for t in range(C // 16):
                dstc[pl.ds(t * 16, 16)] = sdb[1, pl.ds(t * 16, 16)]

            pltpu.make_async_copy(q_hbm.at[sdb.at[1]], qrows, gq).wait()
            pltpu.make_async_copy(k_hbm.at[sdb.at[0]], krows, gk).wait()

            @pl.when(ci + 2 < NCH)
            def _():
                pltpu.async_copy(sd_hbm.at[wid * NCH + ci + 2], sdb, sdm)

            @pl.when(ci + 1 < NCH)
            def _():
                pltpu.make_async_copy(
                    sd_hbm.at[wid * NCH + ci + 1], nsdb, nsdm).wait()
                pltpu.async_copy(q_hbm.at[nsdb.at[1]], nqrows, ngq)
                pltpu.async_copy(k_hbm.at[nsdb.at[0]], nkrows, ngk)

            @pl.when(ci >= 2)
            def _():
                pltpu.make_async_copy(
                    scb, score_hbm.at[pl.ds(base0, C)], st).wait()

            def grp(g, _):
                def dotj(j, scv):
                    i = g * 16 + j
                    a = qrows[i, pl.ds(0, 16)] * krows[i, pl.ds(0, 16)]
                    for dd in range(1, D // 16):
                        a = a + (qrows[i, pl.ds(dd * 16, 16)]
                                 * krows[i, pl.ds(dd * 16, 16)])
                    return jnp.where(lanes == j, jnp.sum(a), scv)

                scv = lax.fori_loop(0, 16, dotj, jnp.zeros((16,), jnp.float32),
                                    unroll=4)
                scb[pl.ds(g * 16, 16)] = scv
                dv = dstc[pl.ds(g * 16, 16)]

                def bodyw(_carry):
                    cur = plsc.load_gather(pm_l, [dv])
                    need = scv > cur
                    plsc.store_scatter(pm_l, [dv], scv, mask=need)
                    cur2 = plsc.load_gather(pm_l, [dv])
                    return jnp.any(scv > cur2)

                lax.while_loop(lambda carry: carry, bodyw, jnp.bool_(True))
                return 0

            lax.fori_loop(0, C // 16, grp, 0)
            pltpu.async_copy(scb, score_hbm.at[pl.ds(base, C)], st)
        return 0

    lax.fori_loop(0, NCH // 2, outer, 0)
    for b in range(2):
        sdb, qrows, krows, scb, gq, gk, st, sdm = bufs[b]
        pltpu.make_async_copy(scb, score_hbm.at[pl.ds(base0, C)], st).wait()
    pltpu.sync_copy(pm_l, pm_hbm.at[wid])


@functools.partial(
    pl.kernel,
    out_type=[
        jax.ShapeDtypeStruct((EP,), jnp.float32),
        jax.ShapeDtypeStruct((NTILES, NP), jnp.float32),
    ],
    mesh=_mesh(),
    compiler_params=_SC_PARAMS,
    scratch_types=[
        pltpu.VMEM((NP,), jnp.float32),
        pltpu.VMEM((NP,), jnp.float32),
        pltpu.VMEM((EPT,), jnp.float32),
        pltpu.VMEM((EPT,), jnp.int32),
        pltpu.VMEM((EPT,), jnp.float32),
        pltpu.VMEM((NTILES, NP // 16), jnp.float32),
        pltpu.VMEM((NP // 16,), jnp.float32),
        pltpu.VMEM_SHARED((NP,), jnp.float32),
        pltpu.SemaphoreType.DMA,
        pltpu.SemaphoreType.DMA,
        pltpu.SemaphoreType.DMA,
    ],
)
def _s2(score_hbm, dst_hbm, pm_hbm, e_hbm, ps_hbm,
        m_l, s_l, scb, dstb, eb, pmb, mslice, m_sh, pma, sca, dsa):
    cc = lax.axis_index("c")
    ss = lax.axis_index("s")
    wid = ss * 2 + cc
    base0 = wid * EPT
    npt = NP // 16

    pltpu.async_copy(
        pm_hbm.at[pl.ds(0, NTILES), pl.ds(ss * npt, npt)], pmb, pma)
    pltpu.async_copy(score_hbm.at[pl.ds(base0, EPT)], scb, sca)
    pltpu.async_copy(dst_hbm.at[pl.ds(base0, EPT)], dstb, dsa)

    zero = jnp.zeros((16,), jnp.float32)

    def zz(i, _):
        s_l[pl.ds(i * 16, 16)] = zero
        return 0

    lax.fori_loop(0, NP // 16, zz, 0)

    pltpu.make_async_copy(
        pm_hbm.at[pl.ds(0, NTILES), pl.ds(ss * npt, npt)], pmb, pma).wait()

    def redg(g, _):
        acc = pmb[0, pl.ds(g * 16, 16)]
        for r in range(1, NTILES):
            acc = jnp.maximum(acc, pmb[r, pl.ds(g * 16, 16)])
        mslice[pl.ds(g * 16, 16)] = acc
        return 0

    lax.fori_loop(0, npt // 16, redg, 0)
    pltpu.sync_copy(mslice, m_sh.at[pl.ds(ss * npt, npt)])
    plsc.subcore_barrier()
    pltpu.sync_copy(m_sh, m_l)

    pltpu.make_async_copy(score_hbm.at[pl.ds(base0, EPT)], scb, sca).wait()
    pltpu.make_async_copy(dst_hbm.at[pl.ds(base0, EPT)], dstb, dsa).wait()

    def grp(i, _):
        dv = dstb[pl.ds(i * 16, 16)]
        sv = scb[pl.ds(i * 16, 16)]
        mv = plsc.load_gather(m_l, [dv])
        ev = jnp.exp(sv - mv)
        eb[pl.ds(i * 16, 16)] = ev
        plsc.addupdate_scatter(s_l, [dv], ev)
        return 0

    lax.fori_loop(0, EPT // 16, grp, 0)

    pltpu.sync_copy(eb, e_hbm.at[pl.ds(base0, EPT)])
    pltpu.sync_copy(s_l, ps_hbm.at[wid])


@functools.partial(
    pl.kernel,
    out_type=jax.ShapeDtypeStruct((2, NP, D), jnp.float32),
    mesh=_mesh(),
    compiler_params=_SC_PARAMS,
    scratch_types=[
        pltpu.VMEM((2, C), jnp.int32),
        pltpu.VMEM((2, C), jnp.int32),
        pltpu.VMEM((C, D), jnp.float32),
        pltpu.VMEM((C, D), jnp.float32),
        pltpu.VMEM((C,), jnp.float32),
        pltpu.VMEM((C,), jnp.float32),
        pltpu.VMEM((C,), jnp.int32),
        pltpu.VMEM((C,), jnp.int32),
        pltpu.VMEM((NP,), jnp.float32),
        pltpu.VMEM_SHARED((NP, D), jnp.float32),
        pltpu.SemaphoreType.DMA,
        pltpu.SemaphoreType.DMA,
        pltpu.SemaphoreType.DMA,
        pltpu.SemaphoreType.DMA,
        pltpu.SemaphoreType.DMA,
        pltpu.SemaphoreType.DMA,
        pltpu.SemaphoreType.DMA,
        pltpu.SemaphoreType.DMA,
    ],
)
def _s3(e_hbm, s_hbm, sd_hbm, v_hbm, aggp_hbm,
        sd0, sd1, vr0, vr1, ab0, ab1, dc0, dc1, s_l, agg,
        gv0, gv1, sc0, sc1, sdm0, sdm1, em0, em1):
    cc = lax.axis_index("c")
    ss = lax.axis_index("s")
    wid = ss * 2 + cc
    base0 = wid * EPT
    rows_per_tile = NP // 16

    pltpu.sync_copy(s_hbm, s_l)

    zero = jnp.zeros((16,), jnp.float32)

    def zr(i, _):
        for dd in range(D // 16):
            vr0[i, pl.ds(dd * 16, 16)] = zero
        return 0

    lax.fori_loop(0, C, zr, 0)
    for jj in range(rows_per_tile // C):
        pltpu.sync_copy(vr0, agg.at[pl.ds(ss * rows_per_tile + jj * C, C)])
    plsc.subcore_barrier()

    bufs = ((sd0, vr0, ab0, dc0, gv0, sc0, sdm0, em0),
            (sd1, vr1, ab1, dc1, gv1, sc1, sdm1, em1))

    pltpu.sync_copy(sd_hbm.at[wid * NCH], sd0)
    pltpu.async_copy(sd_hbm.at[wid * NCH + 1], sd1, sdm1)
    pltpu.sync_copy(e_hbm.at[pl.ds(base0, C)], ab0)
    pltpu.async_copy(e_hbm.at[pl.ds(base0 + C, C)], ab1, em1)
    pltpu.async_copy(v_hbm.at[sd0.at[0]], vr0, gv0)

    def outer(g2, _):
        for b in range(2):
            sdb, vrows, ab, dc, gv, sc, sdm, em = bufs[b]
            nsdb, nvrows, nab, ndc, ngv, nsc, nsdm, nem = bufs[1 - b]
            ci = g2 * 2 + b
            base = base0 + ci * C

            for t in range(C // 16):
                dc[pl.ds(t * 16, 16)] = sdb[1, pl.ds(t * 16, 16)]

            pltpu.make_async_copy(v_hbm.at[sdb.at[0]], vrows, gv).wait()

            @pl.when(ci + 2 < NCH)
            def _():
                pltpu.async_copy(sd_hbm.at[wid * NCH + ci + 2], sdb, sdm)

            @pl.when((ci + 1 < NCH) & (ci >= 1))
            def _():
                pltpu.make_async_copy(nvrows, agg.at[ndc], nsc).wait()

            @pl.when(ci + 1 < NCH)
            def _():
                pltpu.make_async_copy(
                    sd_hbm.at[wid * NCH + ci + 1], nsdb, nsdm).wait()
                pltpu.async_copy(v_hbm.at[nsdb.at[0]], nvrows, ngv)

            @pl.when(ci >= 1)
            def _():
                pltpu.make_async_copy(
                    e_hbm.at[pl.ds(base0, C)], ab, em).wait()

            def grp(g, _):
                dv = dc[pl.ds(g * 16, 16)]
                sv = plsc.load_gather(s_l, [dv])
                av = ab[pl.ds(g * 16, 16)] / sv
                for j in range(16):
                    i = g * 16 + j
                    a = av[j]
                    for dd in range(D // 16):
                        vrows[i, pl.ds(dd * 16, 16)] = (
                            vrows[i, pl.ds(dd * 16, 16)] * a)
                return 0

            lax.fori_loop(0, C // 16, grp, 0)
            pltpu.async_copy(vrows, agg.at[dc], sc, add=True)

            @pl.when(ci + 2 < NCH)
            def _():
                pltpu.async_copy(
                    e_hbm.at[pl.ds(base0 + (ci + 2) * C, C)], ab, em)
        return 0

    lax.fori_loop(0, NCH // 2, outer, 0)
    for b in range(2):
        sdb, vrows, ab, dc, gv, sc, sdm, em = bufs[b]
        pltpu.make_async_copy(vrows, agg.at[dc], sc).wait()
    plsc.subcore_barrier()
    pltpu.sync_copy(
        agg.at[pl.ds(ss * rows_per_tile, rows_per_tile)],
        aggp_hbm.at[cc, pl.ds(ss * rows_per_tile, rows_per_tile)],
    )


def kernel(x, edge_index, W_proj, b_proj, g1, b1, Wq1, Wk1, Wv1, tau1,
           Wq2, Wk2, Wv2, tau2, Wq3, Wk3, Wv3, tau3, W_ctx, b_ctx, g2, b2):
    f32 = jnp.float32
    i32 = jnp.int32

    src0 = edge_index[0].astype(i32)
    dst0 = edge_index[1].astype(i32)
    loop = jnp.arange(N, dtype=i32)
    n_epad = EP - E - N
    padi = N + (jnp.arange(n_epad, dtype=i32) % (NP - N))
    src = jnp.concatenate([src0, loop, padi])
    dst = jnp.concatenate([dst0, loop, padi])
    sd = jnp.stack(
        [src.reshape(NTILES * NCH, C), dst.reshape(NTILES * NCH, C)], axis=1)

    xp = jnp.pad(x.astype(f32), ((0, NP - N), (0, 0)))

    def row(a):
        return a.astype(f32).reshape(1, -1)

    h0 = _proj_ln(xp, W_proj.astype(f32), row(b_proj), row(g1), row(b1))

    layers = [
        (Wq1, Wk1, Wv1, tau1),
        (Wq2, Wk2, Wv2, tau2),
        (Wq3, Wk3, Wv3, tau3),
    ]

    hs = [h0]
    aggp = None
    for (wq, wk, wv, tau) in layers:
        tau2d = tau.astype(f32).reshape(1, 1)
        if aggp is None:
            q, k, v = _qkv1(tau2d, h0, wq.astype(f32), wk.astype(f32),
                            wv.astype(f32))
        else:
            hprev, q, k, v = _qkv2(tau2d, aggp[0], aggp[1], wq.astype(f32),
                                   wk.astype(f32), wv.astype(f32))
            hs.append(hprev)
        score, pm = _s1(q, k, sd)
        e, ps = _s2(score, dst, pm)
        s = _colreduce(ps, _colsum_body).reshape(NP)
        aggp = _s3(e, s, sd, v)

    out = _final(hs[0], hs[1], hs[2], aggp[0], aggp[1], W_ctx.astype(f32),
                 row(b_ctx), row(g2), row(b2))
    return out[:N]
